# Initial kernel scaffold; baseline (speedup 1.0000x reference)
import jax
import jax.numpy as jnp
from jax.experimental import pallas as pl


def kernel(x_paper, x_author, x_term, edge_index_paper__author, edge_index_author__paper, edge_index_paper__term, edge_index_term__paper, c1_Wk_paper, c1_Wq_paper, c1_Wv_paper, c1_bk_paper, c1_bq_paper, c1_bv_paper, c1_Wa_paper, c1_ba_paper, c1_Wk_author, c1_Wq_author, c1_Wv_author, c1_bk_author, c1_bq_author, c1_bv_author, c1_Wa_author, c1_ba_author, c1_Wk_term, c1_Wq_term, c1_Wv_term, c1_bk_term, c1_bq_term, c1_bv_term, c1_Wa_term, c1_ba_term, c1_arel_paper__author, c1_mrel_paper__author, c1_prel_paper__author, c1_arel_author__paper, c1_mrel_author__paper, c1_prel_author__paper, c1_arel_paper__term, c1_mrel_paper__term, c1_prel_paper__term, c1_arel_term__paper, c1_mrel_term__paper, c1_prel_term__paper, c2_Wk_paper, c2_Wq_paper, c2_Wv_paper, c2_bk_paper, c2_bq_paper, c2_bv_paper, c2_Wa_paper, c2_ba_paper, c2_Wk_author, c2_Wq_author, c2_Wv_author, c2_bk_author, c2_bq_author, c2_bv_author, c2_Wa_author, c2_ba_author, c2_Wk_term, c2_Wq_term, c2_Wv_term, c2_bk_term, c2_bq_term, c2_bv_term, c2_Wa_term, c2_ba_term, c2_arel_paper__author, c2_mrel_paper__author, c2_prel_paper__author, c2_arel_author__paper, c2_mrel_author__paper, c2_prel_author__paper, c2_arel_paper__term, c2_mrel_paper__term, c2_prel_paper__term, c2_arel_term__paper, c2_mrel_term__paper, c2_prel_term__paper, lin_W, lin_b):
    raise NotImplementedError("write your pallas kernel here")


# speedup vs baseline: 1.1175x; 1.1175x over previous
import functools

import jax
import jax.numpy as jnp
import numpy as np
from jax.experimental import pallas as pl
from jax.experimental.pallas import tpu as pltpu

NODE_TYPES = ("paper", "author", "term")
N_NODES = {"paper": 4000, "author": 3000, "term": 3000}
EDGE_TYPES = (("paper", "author"), ("author", "paper"), ("paper", "term"), ("term", "paper"))
N_EDGES = 40000
H = 8
DIMS = {1: (2048, 128), 2: (128, 64)}
OUT_DIM = 16


def _blockdiag(rel):
    Hh, dh, _ = rel.shape
    eye = jnp.eye(Hh, dtype=rel.dtype)[:, None, :, None]
    return (eye * rel[:, :, None, :]).reshape(Hh * dh, Hh * dh)


def _layer(xd, ints, p, L):
    din, dout = DIMS[L]
    dh = dout // H
    scale = 1.0 / np.sqrt(dh)
    k = {}
    q = {}
    v = {}
    for t in NODE_TYPES:
        k[t] = xd[t] @ p["c%d_Wk_%s" % (L, t)] + p["c%d_bk_%s" % (L, t)]
        q[t] = xd[t] @ p["c%d_Wq_%s" % (L, t)] + p["c%d_bq_%s" % (L, t)]
        v[t] = xd[t] @ p["c%d_Wv_%s" % (L, t)] + p["c%d_bv_%s" % (L, t)]
    agg = {t: jnp.zeros((N_NODES[t], dout), jnp.float32) for t in NODE_TYPES}
    for (s, d) in EDGE_TYPES:
        en = s + "__" + d
        ei = ints["edge_index_" + en]
        arel = p["c%d_arel_%s" % (L, en)] * (p["c%d_prel_%s" % (L, en)] * scale)[:, None, None]
        A = _blockdiag(arel)
        M = _blockdiag(p["c%d_mrel_%s" % (L, en)])
        krel = k[s] @ A
        vrel = v[s] @ M
        ks_ = krel[ei[0]].reshape(-1, H, dh)
        vs_ = vrel[ei[0]].reshape(-1, H, dh)
        qs_ = q[d][ei[1]].reshape(-1, H, dh)
        ex = jnp.exp((qs_ * ks_).sum(-1))
        num = jax.ops.segment_sum(vs_ * ex[..., None], ei[1], num_segments=N_NODES[d])
        den = jax.ops.segment_sum(ex, ei[1], num_segments=N_NODES[d])
        agg[d] = agg[d] + (num / (den[..., None] + 1e-16)).reshape(N_NODES[d], dout)
    return {
        t: jax.nn.gelu(agg[t]) @ p["c%d_Wa_%s" % (L, t)] + p["c%d_ba_%s" % (L, t)]
        for t in NODE_TYPES
    }


def _final_body(h_ref, w_ref, b_ref, o_ref):
    logits = h_ref[...] @ w_ref[...] + b_ref[...][None, :]
    m = jnp.max(logits, axis=1, keepdims=True)
    e = jnp.exp(logits - m)
    o_ref[...] = e / jnp.sum(e, axis=1, keepdims=True)


@jax.jit
def _final(h, lin_W, lin_b):
    n = h.shape[0]
    return pl.pallas_call(
        _final_body,
        out_shape=jax.ShapeDtypeStruct((n, OUT_DIM), jnp.float32),
        grid=(10,),
        in_specs=[
            pl.BlockSpec((n // 10, 64), lambda i: (i, 0)),
            pl.BlockSpec((64, OUT_DIM), lambda i: (0, 0)),
            pl.BlockSpec((OUT_DIM,), lambda i: (0,)),
        ],
        out_specs=pl.BlockSpec((n // 10, OUT_DIM), lambda i: (i, 0)),
    )(h, lin_W, lin_b)


def kernel(x_paper, x_author, x_term, edge_index_paper__author, edge_index_author__paper, edge_index_paper__term, edge_index_term__paper, c1_Wk_paper, c1_Wq_paper, c1_Wv_paper, c1_bk_paper, c1_bq_paper, c1_bv_paper, c1_Wa_paper, c1_ba_paper, c1_Wk_author, c1_Wq_author, c1_Wv_author, c1_bk_author, c1_bq_author, c1_bv_author, c1_Wa_author, c1_ba_author, c1_Wk_term, c1_Wq_term, c1_Wv_term, c1_bk_term, c1_bq_term, c1_bv_term, c1_Wa_term, c1_ba_term, c1_arel_paper__author, c1_mrel_paper__author, c1_prel_paper__author, c1_arel_author__paper, c1_mrel_author__paper, c1_prel_author__paper, c1_arel_paper__term, c1_mrel_paper__term, c1_prel_paper__term, c1_arel_term__paper, c1_mrel_term__paper, c1_prel_term__paper, c2_Wk_paper, c2_Wq_paper, c2_Wv_paper, c2_bk_paper, c2_bq_paper, c2_bv_paper, c2_Wa_paper, c2_ba_paper, c2_Wk_author, c2_Wq_author, c2_Wv_author, c2_bk_author, c2_bq_author, c2_bv_author, c2_Wa_author, c2_ba_author, c2_Wk_term, c2_Wq_term, c2_Wv_term, c2_bk_term, c2_bq_term, c2_bv_term, c2_Wa_term, c2_ba_term, c2_arel_paper__author, c2_mrel_paper__author, c2_prel_paper__author, c2_arel_author__paper, c2_mrel_author__paper, c2_prel_author__paper, c2_arel_paper__term, c2_mrel_paper__term, c2_prel_paper__term, c2_arel_term__paper, c2_mrel_term__paper, c2_prel_term__paper, lin_W, lin_b):
    kwargs = dict(locals())
    ints = {kk: vv for kk, vv in kwargs.items() if kk.startswith("edge_index_")}
    p = {kk: vv for kk, vv in kwargs.items() if not kk.startswith("edge_index_")}
    xd = {t: p["x_" + t] for t in NODE_TYPES}
    h1 = _layer(xd, ints, p, 1)
    h2 = _layer(h1, ints, p, 2)
    hcat = jnp.vstack([h2[t] for t in NODE_TYPES])
    return _final(hcat, lin_W, lin_b)


# speedup vs baseline: 11.4965x; 10.2878x over previous
import functools

import jax
import jax.numpy as jnp
import numpy as np
from jax import lax
from jax.experimental import pallas as pl
from jax.experimental.pallas import tpu as pltpu
from jax.experimental.pallas import tpu_sc as plsc

NODE_TYPES = ("paper", "author", "term")
N_NODES = {"paper": 4000, "author": 3000, "term": 3000}
N_PAD = {"paper": 4096, "author": 3072, "term": 3072}
EDGE_TYPES = (("paper", "author"), ("author", "paper"), ("paper", "term"), ("term", "paper"))
N_EDGES = 40000
E_PAD = 40960
H = 8
DIMS = {1: (2048, 128), 2: (128, 64)}
OUT_DIM = 16
DP = 128
NW = 32
CHUNK = 64
CHUNKS_PER_W = E_PAD // (NW * CHUNK)


def _prep_body(dh, n_src, *refs):
    wk, wq, wv, bk, bq, bv = refs[:6]
    rels = refs[6:6 + 2 * n_src]
    if dh == 8:
        wa = refs[6 + 2 * n_src]
        wf_ref, bf_ref, wap_ref = refs[7 + 2 * n_src:]
    else:
        wf_ref, bf_ref = refs[6 + 2 * n_src:]

    def padcols(m):
        if dh == 16:
            return m
        return jnp.concatenate([m, jnp.zeros((m.shape[0], 16 - dh), jnp.float32)], axis=1)

    def heads(mat, rel=None):
        cols = []
        for h in range(H):
            blk = mat[:, h * dh:(h + 1) * dh]
            if rel is not None:
                blk = jnp.dot(blk, rel[h], preferred_element_type=jnp.float32)
            cols.append(padcols(blk))
        return jnp.concatenate(cols, axis=1)

    wcols = [heads(wq[...])]
    bcols = [heads(bq[...])]
    for i in range(n_src):
        arel_s = rels[2 * i][...]
        mrel = rels[2 * i + 1][...]
        wcols += [heads(wk[...], arel_s), heads(wv[...], mrel)]
        bcols += [heads(bk[...], arel_s), heads(bv[...], mrel)]
    wf_ref[...] = jnp.concatenate(wcols, axis=1)
    bf_ref[...] = jnp.concatenate(bcols, axis=1)
    if dh == 8:
        wav = wa[...]
        blocks = []
        for h in range(H):
            blocks.append(wav[h * 8:(h + 1) * 8, :])
            blocks.append(jnp.zeros((8, wav.shape[1]), jnp.float32))
        wap_ref[...] = jnp.concatenate(blocks, axis=0)


def _prep(L, t, p, arel_s, mrel):
    din, dout = DIMS[L]
    dh = dout // H
    n_src = len(arel_s)
    K = DP * (1 + 2 * n_src)
    ins = [p["c%d_Wk_%s" % (L, t)], p["c%d_Wq_%s" % (L, t)], p["c%d_Wv_%s" % (L, t)],
           p["c%d_bk_%s" % (L, t)].reshape(1, dout), p["c%d_bq_%s" % (L, t)].reshape(1, dout),
           p["c%d_bv_%s" % (L, t)].reshape(1, dout)]
    for a, m in zip(arel_s, mrel):
        ins += [a, m]
    outs = [jax.ShapeDtypeStruct((din, K), jnp.float32), jax.ShapeDtypeStruct((1, K), jnp.float32)]
    if dh == 8:
        ins.append(p["c%d_Wa_%s" % (L, t)])
        outs.append(jax.ShapeDtypeStruct((DP, dout), jnp.float32))
    return pl.pallas_call(
        functools.partial(_prep_body, dh, n_src),
        out_shape=tuple(outs),
    )(*ins)


def _proj_body(n_kv, x_ref, wf_ref, bf_ref, q_ref, *kv_refs):
    y = jnp.dot(x_ref[...], wf_ref[...], preferred_element_type=jnp.float32) + bf_ref[...]
    q_ref[...] = y[:, :DP]
    for i in range(n_kv):
        kv_refs[i][...] = y[:, DP + 2 * DP * i:DP + 2 * DP * (i + 1)]


def _proj(x, wf, bf, n_kv):
    n, din = x.shape
    K = wf.shape[1]
    R = 1000
    grid = n // R
    return pl.pallas_call(
        functools.partial(_proj_body, n_kv),
        out_shape=tuple([jax.ShapeDtypeStruct((n, DP), jnp.float32)]
                        + [jax.ShapeDtypeStruct((n, 2 * DP), jnp.float32)] * n_kv),
        grid=(grid,),
        in_specs=[
            pl.BlockSpec((R, din), lambda i: (i, 0)),
            pl.BlockSpec((din, K), lambda i: (0, 0)),
            pl.BlockSpec((1, K), lambda i: (0, 0)),
        ],
        out_specs=tuple([pl.BlockSpec((R, DP), lambda i: (i, 0))]
                        + [pl.BlockSpec((R, 2 * DP), lambda i: (i, 0))] * n_kv),
    )(x, wf, bf)


def _edge_body(meta, *refs):
    kvs = refs[0:4]
    qs = refs[4:7]
    idx = refs[7:15]
    zer = refs[15]
    outs = refs[16:20]
    srcb, dstb, dstg, kvb, qb, exb, wvb, sh_den, sh_num, sem, sem2 = refs[20:]
    c = lax.axis_index("c")
    s = lax.axis_index("s")
    w = s * 2 + c
    lane = lax.broadcasted_iota(jnp.int32, (16,), 0)
    gdn = lax.GatherDimensionNumbers(offset_dims=(), collapsed_slice_dims=(0,), start_index_map=(0,))
    shuf_idx = [(lane ^ off).reshape(16, 1) for off in (8, 4, 2, 1)]

    def allsum(x):
        for idx in shuf_idx:
            x = x + lax.gather(x, idx, gdn, (1,),
                               mode=lax.GatherScatterMode.PROMISE_IN_BOUNDS)
        return x
    for ti, (qi, nd, ndp) in enumerate(meta):
        kv = kvs[ti]
        q = qs[qi]
        se, de = idx[2 * ti], idx[2 * ti + 1]
        out = outs[ti]
        rows = ndp // 16
        r0 = s * rows
        pltpu.sync_copy(zer.at[pl.ds(r0, rows)], sh_den.at[pl.ds(r0, rows)])
        pltpu.sync_copy(zer.at[pl.ds(r0, rows)], sh_num.at[pl.ds(r0, rows)])
        plsc.subcore_barrier()

        def chunk(g, _):
            base = (w * CHUNKS_PER_W + g) * CHUNK
            pltpu.sync_copy(se.at[pl.ds(base, CHUNK)], srcb)
            pltpu.sync_copy(de.at[pl.ds(base, CHUNK)], dstb)
            for i in range(CHUNK // 16):
                dstg[pl.ds(i * 16, 16)] = jnp.minimum(dstb[pl.ds(i * 16, 16)], nd - 1)
            pltpu.async_copy(kv.at[srcb], kvb, sem).wait()
            pltpu.async_copy(q.at[dstg], qb, sem2).wait()

            def edge(e, _):
                for h in range(H):
                    kvv = kvb[e, pl.ds(h * 16, 16)]
                    qv = qb[e, pl.ds(h * 16, 16)]
                    exv = jnp.exp(allsum(qv * kvv))
                    exb[e, pl.ds(h * 16, 16)] = exv
                    wvb[e, pl.ds(h * 16, 16)] = exv * kvb[e, pl.ds(DP + h * 16, 16)]
                return 0

            lax.fori_loop(0, CHUNK, edge, 0)
            pltpu.sync_copy(exb, sh_den.at[dstb], add=True)
            pltpu.sync_copy(wvb, sh_num.at[dstb], add=True)
            return 0

        lax.fori_loop(0, CHUNKS_PER_W, chunk, 0)
        plsc.subcore_barrier()
        pltpu.sync_copy(sh_den.at[pl.ds(r0, rows)], out.at[c, 0, pl.ds(r0, rows)])
        pltpu.sync_copy(sh_num.at[pl.ds(r0, rows)], out.at[c, 1, pl.ds(r0, rows)])
        plsc.subcore_barrier()


def _edge_stage(kv, q, srcp, dstp, zer):
    meta = []
    ins = []
    for (s, d) in EDGE_TYPES:
        en = s + "__" + d
        ins.append(kv[en])
        meta.append((NODE_TYPES.index(d), N_NODES[d], N_PAD[d]))
    for t in NODE_TYPES:
        ins.append(q[t])
    for (s, d) in EDGE_TYPES:
        en = s + "__" + d
        ins += [srcp[en], dstp[en]]
    ins.append(zer)
    out_type = tuple(jax.ShapeDtypeStruct((2, 2, N_PAD[d], DP), jnp.float32) for (s, d) in EDGE_TYPES)
    mesh = plsc.VectorSubcoreMesh(core_axis_name="c", subcore_axis_name="s")
    f = pl.kernel(
        functools.partial(_edge_body, meta),
        out_type=out_type,
        mesh=mesh,
        scratch_types=[
            pltpu.VMEM((CHUNK,), jnp.int32),
            pltpu.VMEM((CHUNK,), jnp.int32),
            pltpu.VMEM((CHUNK,), jnp.int32),
            pltpu.VMEM((CHUNK, 2 * DP), jnp.float32),
            pltpu.VMEM((CHUNK, DP), jnp.float32),
            pltpu.VMEM((CHUNK, DP), jnp.float32),
            pltpu.VMEM((CHUNK, DP), jnp.float32),
            pltpu.VMEM_SHARED((N_PAD["paper"], DP), jnp.float32),
            pltpu.VMEM_SHARED((N_PAD["paper"], DP), jnp.float32),
            pltpu.SemaphoreType.DMA,
            pltpu.SemaphoreType.DMA,
        ],
    )
    return f(*ins)


def _post_body(n_en, slab0, *refs):
    slabs = (slab0,) + refs[:n_en - 1]
    wa_ref, ba_ref, out_ref = refs[n_en - 1:]
    agg = jnp.zeros_like(slabs[0][0, 0])
    for sl_ref in slabs:
        sl = sl_ref[...]
        den = sl[0, 0] + sl[1, 0]
        num = sl[0, 1] + sl[1, 1]
        agg = agg + num / (den + 1e-16)
    g = 0.5 * agg * (1.0 + jnp.tanh(0.7978845608028654 * (agg + 0.044715 * agg * agg * agg)))
    out_ref[...] = jnp.dot(g, wa_ref[...], preferred_element_type=jnp.float32) + ba_ref[...]


def _post(slabs, wa, ba, n):
    n_en = len(slabs)
    dout = wa.shape[1]
    R = 1000
    grid = n // R
    return pl.pallas_call(
        functools.partial(_post_body, n_en),
        out_shape=jax.ShapeDtypeStruct((n, dout), jnp.float32),
        grid=(grid,),
        in_specs=[pl.BlockSpec((2, 2, R, DP), lambda i: (0, 0, i, 0))] * n_en
        + [pl.BlockSpec((DP, dout), lambda i: (0, 0)),
           pl.BlockSpec((1, dout), lambda i: (0, 0))],
        out_specs=pl.BlockSpec((R, dout), lambda i: (i, 0)),
    )(*slabs, wa, ba.reshape(1, dout))


def _final_body(h_ref, w_ref, b_ref, o_ref):
    logits = jnp.dot(h_ref[...], w_ref[...], preferred_element_type=jnp.float32) + b_ref[...]
    m = jnp.max(logits, axis=1, keepdims=True)
    e = jnp.exp(logits - m)
    o_ref[...] = e / jnp.sum(e, axis=1, keepdims=True)


def _final(h, lin_W, lin_b):
    n = h.shape[0]
    return pl.pallas_call(
        _final_body,
        out_shape=jax.ShapeDtypeStruct((n, OUT_DIM), jnp.float32),
        grid=(10,),
        in_specs=[
            pl.BlockSpec((n // 10, 64), lambda i: (i, 0)),
            pl.BlockSpec((64, OUT_DIM), lambda i: (0, 0)),
            pl.BlockSpec((1, OUT_DIM), lambda i: (0, 0)),
        ],
        out_specs=pl.BlockSpec((n // 10, OUT_DIM), lambda i: (i, 0)),
    )(h, lin_W, lin_b.reshape(1, OUT_DIM))


def _layer(xd, srcp, dstp, zer, p, L):
    din, dout = DIMS[L]
    dh = dout // H
    scale = 1.0 / np.sqrt(dh)
    src_of = {t: [en for en in EDGE_TYPES if en[0] == t] for t in NODE_TYPES}
    dst_of = {t: [en for en in EDGE_TYPES if en[1] == t] for t in NODE_TYPES}
    q = {}
    kv = {}
    wa_pad = {}
    for t in NODE_TYPES:
        arel_s = []
        mrel = []
        for (s, d) in src_of[t]:
            en = s + "__" + d
            arel_s.append(p["c%d_arel_%s" % (L, en)]
                          * (p["c%d_prel_%s" % (L, en)] * scale)[:, None, None])
            mrel.append(p["c%d_mrel_%s" % (L, en)])
        pr = _prep(L, t, p, arel_s, mrel)
        wf, bf = pr[0], pr[1]
        if dh == 8:
            wa_pad[t] = pr[2]
        outs = _proj(xd[t], wf, bf, len(src_of[t]))
        q[t] = outs[0]
        for i, (s, d) in enumerate(src_of[t]):
            kv[s + "__" + d] = outs[1 + i]
    slabs = _edge_stage(kv, q, srcp, dstp, zer)
    slab_of = {s + "__" + d: slabs[i] for i, (s, d) in enumerate(EDGE_TYPES)}
    out = {}
    for t in NODE_TYPES:
        sl = [slab_of[s + "__" + d] for (s, d) in dst_of[t]]
        wa = wa_pad[t] if dh == 8 else p["c%d_Wa_%s" % (L, t)]
        out[t] = _post(sl, wa, p["c%d_ba_%s" % (L, t)], N_NODES[t])
    return out


def kernel(x_paper, x_author, x_term, edge_index_paper__author, edge_index_author__paper, edge_index_paper__term, edge_index_term__paper, c1_Wk_paper, c1_Wq_paper, c1_Wv_paper, c1_bk_paper, c1_bq_paper, c1_bv_paper, c1_Wa_paper, c1_ba_paper, c1_Wk_author, c1_Wq_author, c1_Wv_author, c1_bk_author, c1_bq_author, c1_bv_author, c1_Wa_author, c1_ba_author, c1_Wk_term, c1_Wq_term, c1_Wv_term, c1_bk_term, c1_bq_term, c1_bv_term, c1_Wa_term, c1_ba_term, c1_arel_paper__author, c1_mrel_paper__author, c1_prel_paper__author, c1_arel_author__paper, c1_mrel_author__paper, c1_prel_author__paper, c1_arel_paper__term, c1_mrel_paper__term, c1_prel_paper__term, c1_arel_term__paper, c1_mrel_term__paper, c1_prel_term__paper, c2_Wk_paper, c2_Wq_paper, c2_Wv_paper, c2_bk_paper, c2_bq_paper, c2_bv_paper, c2_Wa_paper, c2_ba_paper, c2_Wk_author, c2_Wq_author, c2_Wv_author, c2_bk_author, c2_bq_author, c2_bv_author, c2_Wa_author, c2_ba_author, c2_Wk_term, c2_Wq_term, c2_Wv_term, c2_bk_term, c2_bq_term, c2_bv_term, c2_Wa_term, c2_ba_term, c2_arel_paper__author, c2_mrel_paper__author, c2_prel_paper__author, c2_arel_author__paper, c2_mrel_author__paper, c2_prel_author__paper, c2_arel_paper__term, c2_mrel_paper__term, c2_prel_paper__term, c2_arel_term__paper, c2_mrel_term__paper, c2_prel_term__paper, lin_W, lin_b):
    kwargs = dict(locals())
    p = {kk: vv for kk, vv in kwargs.items() if not kk.startswith("edge_index_")}
    srcp = {}
    dstp = {}
    for (s, d) in EDGE_TYPES:
        en = s + "__" + d
        ei = kwargs["edge_index_" + en]
        pad = E_PAD - N_EDGES
        srcp[en] = jnp.concatenate([ei[0], jnp.zeros((pad,), ei.dtype)])
        dstp[en] = jnp.concatenate([ei[1], jnp.full((pad,), N_NODES[d], ei.dtype)])
    zer = jnp.zeros((N_PAD["paper"], DP), jnp.float32)
    xd = {t: p["x_" + t] for t in NODE_TYPES}
    h1 = _layer(xd, srcp, dstp, zer, p, 1)
    h2 = _layer(h1, srcp, dstp, zer, p, 2)
    hcat = jnp.vstack([h2[t] for t in NODE_TYPES])
    return _final(hcat, lin_W, lin_b)


# speedup vs baseline: 36.2774x; 3.1555x over previous
import functools

import jax
import jax.numpy as jnp
import numpy as np
from jax import lax
from jax.experimental import pallas as pl
from jax.experimental.pallas import tpu as pltpu
from jax.experimental.pallas import tpu_sc as plsc

NODE_TYPES = ("paper", "author", "term")
N_NODES = {"paper": 4000, "author": 3000, "term": 3000}
N_PAD = {"paper": 4096, "author": 3072, "term": 3072}
EDGE_TYPES = (("paper", "author"), ("author", "paper"), ("paper", "term"), ("term", "paper"))
N_EDGES = 40000
E_PAD = 40960
H = 8
DIMS = {1: (2048, 128), 2: (128, 64)}
OUT_DIM = 16
DP = 128
NW = 32
CHUNK = 32
CHUNKS_PER_W = E_PAD // (NW * CHUNK)


def _prep_body(dh, n_src, *refs):
    wk, wq, wv, bk, bq, bv = refs[:6]
    rels = refs[6:6 + 2 * n_src]
    if dh == 8:
        wa = refs[6 + 2 * n_src]
        wf_ref, bf_ref, wap_ref = refs[7 + 2 * n_src:]
    else:
        wf_ref, bf_ref = refs[6 + 2 * n_src:]

    def padcols(m):
        if dh == 16:
            return m
        return jnp.concatenate([m, jnp.zeros((m.shape[0], 16 - dh), jnp.float32)], axis=1)

    def heads(mat, rel=None):
        cols = []
        for h in range(H):
            blk = mat[:, h * dh:(h + 1) * dh]
            if rel is not None:
                blk = jnp.dot(blk, rel[h], preferred_element_type=jnp.float32)
            cols.append(padcols(blk))
        return jnp.concatenate(cols, axis=1)

    wcols = [heads(wq[...])]
    bcols = [heads(bq[...])]
    for i in range(n_src):
        arel_s = rels[2 * i][...]
        mrel = rels[2 * i + 1][...]
        wcols += [heads(wk[...], arel_s), heads(wv[...], mrel)]
        bcols += [heads(bk[...], arel_s), heads(bv[...], mrel)]
    wf_ref[...] = jnp.concatenate(wcols, axis=1)
    bf_ref[...] = jnp.concatenate(bcols, axis=1)
    if dh == 8:
        wav = wa[...]
        blocks = []
        for h in range(H):
            blocks.append(wav[h * 8:(h + 1) * 8, :])
            blocks.append(jnp.zeros((8, wav.shape[1]), jnp.float32))
        wap_ref[...] = jnp.concatenate(blocks, axis=0)


def _prep(L, t, p, arel_s, mrel):
    din, dout = DIMS[L]
    dh = dout // H
    n_src = len(arel_s)
    K = DP * (1 + 2 * n_src)
    ins = [p["c%d_Wk_%s" % (L, t)], p["c%d_Wq_%s" % (L, t)], p["c%d_Wv_%s" % (L, t)],
           p["c%d_bk_%s" % (L, t)].reshape(1, dout), p["c%d_bq_%s" % (L, t)].reshape(1, dout),
           p["c%d_bv_%s" % (L, t)].reshape(1, dout)]
    for a, m in zip(arel_s, mrel):
        ins += [a, m]
    outs = [jax.ShapeDtypeStruct((din, K), jnp.float32), jax.ShapeDtypeStruct((1, K), jnp.float32)]
    if dh == 8:
        ins.append(p["c%d_Wa_%s" % (L, t)])
        outs.append(jax.ShapeDtypeStruct((DP, dout), jnp.float32))
    return pl.pallas_call(
        functools.partial(_prep_body, dh, n_src),
        out_shape=tuple(outs),
    )(*ins)


def _proj_body(n_kv, x_ref, wf_ref, bf_ref, q_ref, *kv_refs):
    y = jnp.dot(x_ref[...], wf_ref[...], preferred_element_type=jnp.float32) + bf_ref[...]
    q_ref[...] = y[:, :DP]
    for i in range(n_kv):
        kv_refs[i][...] = y[:, DP + 2 * DP * i:DP + 2 * DP * (i + 1)]


def _proj(x, wf, bf, n_kv):
    n, din = x.shape
    K = wf.shape[1]
    R = 1000
    grid = n // R
    return pl.pallas_call(
        functools.partial(_proj_body, n_kv),
        out_shape=tuple([jax.ShapeDtypeStruct((n, DP), jnp.float32)]
                        + [jax.ShapeDtypeStruct((n, 2 * DP), jnp.float32)] * n_kv),
        grid=(grid,),
        in_specs=[
            pl.BlockSpec((R, din), lambda i: (i, 0)),
            pl.BlockSpec((din, K), lambda i: (0, 0)),
            pl.BlockSpec((1, K), lambda i: (0, 0)),
        ],
        out_specs=tuple([pl.BlockSpec((R, DP), lambda i: (i, 0))]
                        + [pl.BlockSpec((R, 2 * DP), lambda i: (i, 0))] * n_kv),
    )(x, wf, bf)


def _edge_body(meta, *refs):
    kvs = refs[0:4]
    qs = refs[4:7]
    idx = refs[7:15]
    zer = refs[15]
    outs = refs[16:20]
    (srcb0, dstb0, dstg0, srcb1, dstb1, dstg1, kvb0, kvb1, qb0, qb1,
     exb, wvb, sh_den, sh_num, skv0, skv1, sq0, sq1) = refs[20:]
    c = lax.axis_index("c")
    s = lax.axis_index("s")
    w = s * 2 + c
    lane = lax.broadcasted_iota(jnp.int32, (16,), 0)
    gdn = lax.GatherDimensionNumbers(offset_dims=(), collapsed_slice_dims=(0,), start_index_map=(0,))
    shuf_idx = [(lane ^ off).reshape(16, 1) for off in (8, 4, 2, 1)]

    def allsum(x):
        for idx in shuf_idx:
            x = x + lax.gather(x, idx, gdn, (1,),
                               mode=lax.GatherScatterMode.PROMISE_IN_BOUNDS)
        return x
    for ti, (qi, nd, ndp) in enumerate(meta):
        kv = kvs[ti]
        q = qs[qi]
        se, de = idx[2 * ti], idx[2 * ti + 1]
        out = outs[ti]
        rows = ndp // 16
        r0 = s * rows
        pltpu.sync_copy(zer.at[pl.ds(r0, rows)], sh_den.at[pl.ds(r0, rows)])
        pltpu.sync_copy(zer.at[pl.ds(r0, rows)], sh_num.at[pl.ds(r0, rows)])
        plsc.subcore_barrier()

        def fire(g, sb, db, dg, kb, qbf, sk, sq):
            base = (w * CHUNKS_PER_W + g) * CHUNK
            pltpu.sync_copy(se.at[pl.ds(base, CHUNK)], sb)
            pltpu.sync_copy(de.at[pl.ds(base, CHUNK)], db)
            for i in range(CHUNK // 16):
                dg[pl.ds(i * 16, 16)] = jnp.minimum(db[pl.ds(i * 16, 16)], nd - 1)
            pltpu.async_copy(kv.at[sb], kb, sk)
            pltpu.async_copy(q.at[dg], qbf, sq)

        def drain_compute_scatter(sb, db, dg, kb, qbf, sk, sq):
            pltpu.make_async_copy(kv.at[sb], kb, sk).wait()
            pltpu.make_async_copy(q.at[dg], qbf, sq).wait()

            @plsc.parallel_loop(0, CHUNK, 1, unroll=2)
            def edge(e):
                for h in range(H):
                    kvv = kb[e, pl.ds(h * 16, 16)]
                    qv = qbf[e, pl.ds(h * 16, 16)]
                    exv = jnp.exp(allsum(qv * kvv))
                    exb[e, pl.ds(h * 16, 16)] = exv
                    wvb[e, pl.ds(h * 16, 16)] = exv * kb[e, pl.ds(DP + h * 16, 16)]

            pltpu.sync_copy(exb, sh_den.at[db], add=True)
            pltpu.sync_copy(wvb, sh_num.at[db], add=True)

        bufs0 = (srcb0, dstb0, dstg0, kvb0, qb0, skv0, sq0)
        bufs1 = (srcb1, dstb1, dstg1, kvb1, qb1, skv1, sq1)
        pairs = CHUNKS_PER_W // 2
        fire(0, *bufs0)

        def pair(p, _):
            fire(2 * p + 1, *bufs1)
            drain_compute_scatter(*bufs0)

            @pl.when(p < pairs - 1)
            def _():
                fire(2 * p + 2, *bufs0)

            drain_compute_scatter(*bufs1)
            return 0

        lax.fori_loop(0, pairs, pair, 0)
        plsc.subcore_barrier()
        pltpu.sync_copy(sh_den.at[pl.ds(r0, rows)], out.at[c, 0, pl.ds(r0, rows)])
        pltpu.sync_copy(sh_num.at[pl.ds(r0, rows)], out.at[c, 1, pl.ds(r0, rows)])
        plsc.subcore_barrier()


def _edge_stage(kv, q, srcp, dstp, zer):
    meta = []
    ins = []
    for (s, d) in EDGE_TYPES:
        en = s + "__" + d
        ins.append(kv[en])
        meta.append((NODE_TYPES.index(d), N_NODES[d], N_PAD[d]))
    for t in NODE_TYPES:
        ins.append(q[t])
    for (s, d) in EDGE_TYPES:
        en = s + "__" + d
        ins += [srcp[en], dstp[en]]
    ins.append(zer)
    out_type = tuple(jax.ShapeDtypeStruct((2, 2, N_PAD[d], DP), jnp.float32) for (s, d) in EDGE_TYPES)
    mesh = plsc.VectorSubcoreMesh(core_axis_name="c", subcore_axis_name="s")
    f = pl.kernel(
        functools.partial(_edge_body, meta),
        out_type=out_type,
        mesh=mesh,
        scratch_types=[pltpu.VMEM((CHUNK,), jnp.int32)] * 6
        + [pltpu.VMEM((CHUNK, 2 * DP), jnp.float32)] * 2
        + [pltpu.VMEM((CHUNK, DP), jnp.float32)] * 4
        + [pltpu.VMEM_SHARED((N_PAD["paper"], DP), jnp.float32)] * 2
        + [pltpu.SemaphoreType.DMA] * 4,
    )
    return f(*ins)


def _post_body(n_en, slab0, *refs):
    slabs = (slab0,) + refs[:n_en - 1]
    wa_ref, ba_ref, out_ref = refs[n_en - 1:]
    agg = jnp.zeros_like(slabs[0][0, 0])
    for sl_ref in slabs:
        sl = sl_ref[...]
        den = sl[0, 0] + sl[1, 0]
        num = sl[0, 1] + sl[1, 1]
        agg = agg + num / (den + 1e-16)
    g = 0.5 * agg * (1.0 + jnp.tanh(0.7978845608028654 * (agg + 0.044715 * agg * agg * agg)))
    out_ref[...] = jnp.dot(g, wa_ref[...], preferred_element_type=jnp.float32) + ba_ref[...]


def _post(slabs, wa, ba, n):
    n_en = len(slabs)
    dout = wa.shape[1]
    R = 1000
    grid = n // R
    return pl.pallas_call(
        functools.partial(_post_body, n_en),
        out_shape=jax.ShapeDtypeStruct((n, dout), jnp.float32),
        grid=(grid,),
        in_specs=[pl.BlockSpec((2, 2, R, DP), lambda i: (0, 0, i, 0))] * n_en
        + [pl.BlockSpec((DP, dout), lambda i: (0, 0)),
           pl.BlockSpec((1, dout), lambda i: (0, 0))],
        out_specs=pl.BlockSpec((R, dout), lambda i: (i, 0)),
    )(*slabs, wa, ba.reshape(1, dout))


def _final_body(h_ref, w_ref, b_ref, o_ref):
    logits = jnp.dot(h_ref[...], w_ref[...], preferred_element_type=jnp.float32) + b_ref[...]
    m = jnp.max(logits, axis=1, keepdims=True)
    e = jnp.exp(logits - m)
    o_ref[...] = e / jnp.sum(e, axis=1, keepdims=True)


def _final(h, lin_W, lin_b):
    n = h.shape[0]
    return pl.pallas_call(
        _final_body,
        out_shape=jax.ShapeDtypeStruct((n, OUT_DIM), jnp.float32),
        grid=(10,),
        in_specs=[
            pl.BlockSpec((n // 10, 64), lambda i: (i, 0)),
            pl.BlockSpec((64, OUT_DIM), lambda i: (0, 0)),
            pl.BlockSpec((1, OUT_DIM), lambda i: (0, 0)),
        ],
        out_specs=pl.BlockSpec((n // 10, OUT_DIM), lambda i: (i, 0)),
    )(h, lin_W, lin_b.reshape(1, OUT_DIM))


def _layer(xd, srcp, dstp, zer, p, L):
    din, dout = DIMS[L]
    dh = dout // H
    scale = 1.0 / np.sqrt(dh)
    src_of = {t: [en for en in EDGE_TYPES if en[0] == t] for t in NODE_TYPES}
    dst_of = {t: [en for en in EDGE_TYPES if en[1] == t] for t in NODE_TYPES}
    q = {}
    kv = {}
    wa_pad = {}
    for t in NODE_TYPES:
        arel_s = []
        mrel = []
        for (s, d) in src_of[t]:
            en = s + "__" + d
            arel_s.append(p["c%d_arel_%s" % (L, en)]
                          * (p["c%d_prel_%s" % (L, en)] * scale)[:, None, None])
            mrel.append(p["c%d_mrel_%s" % (L, en)])
        pr = _prep(L, t, p, arel_s, mrel)
        wf, bf = pr[0], pr[1]
        if dh == 8:
            wa_pad[t] = pr[2]
        outs = _proj(xd[t], wf, bf, len(src_of[t]))
        q[t] = outs[0]
        for i, (s, d) in enumerate(src_of[t]):
            kv[s + "__" + d] = outs[1 + i]
    slabs = _edge_stage(kv, q, srcp, dstp, zer)
    slab_of = {s + "__" + d: slabs[i] for i, (s, d) in enumerate(EDGE_TYPES)}
    out = {}
    for t in NODE_TYPES:
        sl = [slab_of[s + "__" + d] for (s, d) in dst_of[t]]
        wa = wa_pad[t] if dh == 8 else p["c%d_Wa_%s" % (L, t)]
        out[t] = _post(sl, wa, p["c%d_ba_%s" % (L, t)], N_NODES[t])
    return out


def kernel(x_paper, x_author, x_term, edge_index_paper__author, edge_index_author__paper, edge_index_paper__term, edge_index_term__paper, c1_Wk_paper, c1_Wq_paper, c1_Wv_paper, c1_bk_paper, c1_bq_paper, c1_bv_paper, c1_Wa_paper, c1_ba_paper, c1_Wk_author, c1_Wq_author, c1_Wv_author, c1_bk_author, c1_bq_author, c1_bv_author, c1_Wa_author, c1_ba_author, c1_Wk_term, c1_Wq_term, c1_Wv_term, c1_bk_term, c1_bq_term, c1_bv_term, c1_Wa_term, c1_ba_term, c1_arel_paper__author, c1_mrel_paper__author, c1_prel_paper__author, c1_arel_author__paper, c1_mrel_author__paper, c1_prel_author__paper, c1_arel_paper__term, c1_mrel_paper__term, c1_prel_paper__term, c1_arel_term__paper, c1_mrel_term__paper, c1_prel_term__paper, c2_Wk_paper, c2_Wq_paper, c2_Wv_paper, c2_bk_paper, c2_bq_paper, c2_bv_paper, c2_Wa_paper, c2_ba_paper, c2_Wk_author, c2_Wq_author, c2_Wv_author, c2_bk_author, c2_bq_author, c2_bv_author, c2_Wa_author, c2_ba_author, c2_Wk_term, c2_Wq_term, c2_Wv_term, c2_bk_term, c2_bq_term, c2_bv_term, c2_Wa_term, c2_ba_term, c2_arel_paper__author, c2_mrel_paper__author, c2_prel_paper__author, c2_arel_author__paper, c2_mrel_author__paper, c2_prel_author__paper, c2_arel_paper__term, c2_mrel_paper__term, c2_prel_paper__term, c2_arel_term__paper, c2_mrel_term__paper, c2_prel_term__paper, lin_W, lin_b):
    kwargs = dict(locals())
    p = {kk: vv for kk, vv in kwargs.items() if not kk.startswith("edge_index_")}
    srcp = {}
    dstp = {}
    for (s, d) in EDGE_TYPES:
        en = s + "__" + d
        ei = kwargs["edge_index_" + en]
        pad = E_PAD - N_EDGES
        srcp[en] = jnp.concatenate([ei[0], jnp.zeros((pad,), ei.dtype)])
        dstp[en] = jnp.concatenate([ei[1], jnp.full((pad,), N_NODES[d], ei.dtype)])
    zer = jnp.zeros((N_PAD["paper"], DP), jnp.float32)
    xd = {t: p["x_" + t] for t in NODE_TYPES}
    h1 = _layer(xd, srcp, dstp, zer, p, 1)
    h2 = _layer(h1, srcp, dstp, zer, p, 2)
    hcat = jnp.vstack([h2[t] for t in NODE_TYPES])
    return _final(hcat, lin_W, lin_b)


# speedup vs baseline: 40.2792x; 1.1103x over previous
import functools

import jax
import jax.numpy as jnp
import numpy as np
from jax import lax
from jax.experimental import pallas as pl
from jax.experimental.pallas import tpu as pltpu
from jax.experimental.pallas import tpu_sc as plsc

NODE_TYPES = ("paper", "author", "term")
N_NODES = {"paper": 4000, "author": 3000, "term": 3000}
N_PAD = {"paper": 4096, "author": 3072, "term": 3072}
EDGE_TYPES = (("paper", "author"), ("author", "paper"), ("paper", "term"), ("term", "paper"))
N_EDGES = 40000
E_PAD = 40960
H = 8
DIMS = {1: (2048, 128), 2: (128, 64)}
OUT_DIM = 16
DP = 128
NW = 32
CHUNK = 32
CHUNKS_PER_W = E_PAD // (NW * CHUNK)


def _prep_body(dh, n_src, *refs):
    wk, wq, wv, bk, bq, bv = refs[:6]
    rels = refs[6:6 + 2 * n_src]
    if dh == 8:
        wa = refs[6 + 2 * n_src]
        wf_ref, bf_ref, wap_ref = refs[7 + 2 * n_src:]
    else:
        wf_ref, bf_ref = refs[6 + 2 * n_src:]

    def padcols(m):
        if dh == 16:
            return m
        return jnp.concatenate([m, jnp.zeros((m.shape[0], 16 - dh), jnp.float32)], axis=1)

    def heads(mat, rel=None):
        cols = []
        for h in range(H):
            blk = mat[:, h * dh:(h + 1) * dh]
            if rel is not None:
                blk = jnp.dot(blk, rel[h], preferred_element_type=jnp.float32)
            cols.append(padcols(blk))
        return jnp.concatenate(cols, axis=1)

    wcols = [heads(wq[...])]
    bcols = [heads(bq[...])]
    for i in range(n_src):
        arel_s = rels[2 * i][...]
        mrel = rels[2 * i + 1][...]
        wcols += [heads(wk[...], arel_s), heads(wv[...], mrel)]
        bcols += [heads(bk[...], arel_s), heads(bv[...], mrel)]
    wf_ref[...] = jnp.concatenate(wcols, axis=1)
    bf_ref[...] = jnp.concatenate(bcols, axis=1)
    if dh == 8:
        wav = wa[...]
        blocks = []
        for h in range(H):
            blocks.append(wav[h * 8:(h + 1) * 8, :])
            blocks.append(jnp.zeros((8, wav.shape[1]), jnp.float32))
        wap_ref[...] = jnp.concatenate(blocks, axis=0)


def _prep(L, t, p, arel_s, mrel):
    din, dout = DIMS[L]
    dh = dout // H
    n_src = len(arel_s)
    K = DP * (1 + 2 * n_src)
    ins = [p["c%d_Wk_%s" % (L, t)], p["c%d_Wq_%s" % (L, t)], p["c%d_Wv_%s" % (L, t)],
           p["c%d_bk_%s" % (L, t)].reshape(1, dout), p["c%d_bq_%s" % (L, t)].reshape(1, dout),
           p["c%d_bv_%s" % (L, t)].reshape(1, dout)]
    for a, m in zip(arel_s, mrel):
        ins += [a, m]
    outs = [jax.ShapeDtypeStruct((din, K), jnp.float32), jax.ShapeDtypeStruct((1, K), jnp.float32)]
    if dh == 8:
        ins.append(p["c%d_Wa_%s" % (L, t)])
        outs.append(jax.ShapeDtypeStruct((DP, dout), jnp.float32))
    return pl.pallas_call(
        functools.partial(_prep_body, dh, n_src),
        out_shape=tuple(outs),
    )(*ins)


def _proj_body(n_kv, x_ref, wf_ref, bf_ref, q_ref, *kv_refs):
    y = jnp.dot(x_ref[...], wf_ref[...], preferred_element_type=jnp.float32) + bf_ref[...]
    q_ref[...] = y[:, :DP]
    for i in range(n_kv):
        kv_refs[i][...] = y[:, DP + 2 * DP * i:DP + 2 * DP * (i + 1)]


def _proj(x, wf, bf, n_kv):
    n, din = x.shape
    K = wf.shape[1]
    R = 1000
    grid = n // R
    return pl.pallas_call(
        functools.partial(_proj_body, n_kv),
        out_shape=tuple([jax.ShapeDtypeStruct((n, DP), jnp.float32)]
                        + [jax.ShapeDtypeStruct((n, 2 * DP), jnp.float32)] * n_kv),
        grid=(grid,),
        in_specs=[
            pl.BlockSpec((R, din), lambda i: (i, 0)),
            pl.BlockSpec((din, K), lambda i: (0, 0)),
            pl.BlockSpec((1, K), lambda i: (0, 0)),
        ],
        out_specs=tuple([pl.BlockSpec((R, DP), lambda i: (i, 0))]
                        + [pl.BlockSpec((R, 2 * DP), lambda i: (i, 0))] * n_kv),
    )(x, wf, bf)


def _edge_body(meta, *refs):
    kvs = refs[0:4]
    qs = refs[4:7]
    idx = refs[7:15]
    zer = refs[15]
    outs = refs[16:20]
    (srcb0, dstb0, dstg0, srcb1, dstb1, dstg1, dsts0, dsts1, kvb0, kvb1, qb0, qb1,
     exb0, exb1, wvb0, wvb1, sh_den, sh_num,
     skv0, skv1, sq0, sq1, sx0, sx1, sw0, sw1) = refs[20:]
    c = lax.axis_index("c")
    s = lax.axis_index("s")
    w = s * 2 + c
    lane = lax.broadcasted_iota(jnp.int32, (16,), 0)
    gdn = lax.GatherDimensionNumbers(offset_dims=(), collapsed_slice_dims=(0,), start_index_map=(0,))
    shuf_idx = [(lane ^ off).reshape(16, 1) for off in (8, 4, 2, 1)]

    def allsum(x):
        for idx in shuf_idx:
            x = x + lax.gather(x, idx, gdn, (1,),
                               mode=lax.GatherScatterMode.PROMISE_IN_BOUNDS)
        return x
    for ti, (qi, nd, ndp) in enumerate(meta):
        kv = kvs[ti]
        q = qs[qi]
        se, de = idx[2 * ti], idx[2 * ti + 1]
        out = outs[ti]
        rows = ndp // 16
        r0 = s * rows
        pltpu.sync_copy(zer.at[pl.ds(r0, rows)], sh_den.at[pl.ds(r0, rows)])
        pltpu.sync_copy(zer.at[pl.ds(r0, rows)], sh_num.at[pl.ds(r0, rows)])
        plsc.subcore_barrier()

        def fire(g, sb, db, dg, kb, qbf, sk, sq):
            base = (w * CHUNKS_PER_W + g) * CHUNK
            pltpu.sync_copy(se.at[pl.ds(base, CHUNK)], sb)
            pltpu.sync_copy(de.at[pl.ds(base, CHUNK)], db)
            for i in range(CHUNK // 16):
                dg[pl.ds(i * 16, 16)] = jnp.minimum(db[pl.ds(i * 16, 16)], nd - 1)
            pltpu.async_copy(kv.at[sb], kb, sk)
            pltpu.async_copy(q.at[dg], qbf, sq)

        def drain_compute_scatter(p, sb, db, dg, kb, qbf, sk, sq, xb, wb, ds_, sx, sw):
            pltpu.make_async_copy(kv.at[sb], kb, sk).wait()
            pltpu.make_async_copy(q.at[dg], qbf, sq).wait()

            @pl.when(p > 0)
            def _():
                pltpu.make_async_copy(xb, sh_den.at[ds_], sx).wait()
                pltpu.make_async_copy(wb, sh_num.at[ds_], sw).wait()

            for i in range(CHUNK // 16):
                ds_[pl.ds(i * 16, 16)] = db[pl.ds(i * 16, 16)]

            @plsc.parallel_loop(0, CHUNK, 1, unroll=2)
            def edge(e):
                for h in range(H):
                    kvv = kb[e, pl.ds(h * 16, 16)]
                    qv = qbf[e, pl.ds(h * 16, 16)]
                    exv = jnp.exp(allsum(qv * kvv))
                    xb[e, pl.ds(h * 16, 16)] = exv
                    wb[e, pl.ds(h * 16, 16)] = exv * kb[e, pl.ds(DP + h * 16, 16)]

            pltpu.async_copy(xb, sh_den.at[ds_], sx, add=True)
            pltpu.async_copy(wb, sh_num.at[ds_], sw, add=True)

        bufs0 = (srcb0, dstb0, dstg0, kvb0, qb0, skv0, sq0)
        bufs1 = (srcb1, dstb1, dstg1, kvb1, qb1, skv1, sq1)
        sc0 = (exb0, wvb0, dsts0, sx0, sw0)
        sc1 = (exb1, wvb1, dsts1, sx1, sw1)
        pairs = CHUNKS_PER_W // 2
        fire(0, *bufs0)

        def pair(p, _):
            fire(2 * p + 1, *bufs1)
            drain_compute_scatter(p, *bufs0, *sc0)

            @pl.when(p < pairs - 1)
            def _():
                fire(2 * p + 2, *bufs0)

            drain_compute_scatter(p, *bufs1, *sc1)
            return 0

        lax.fori_loop(0, pairs, pair, 0)
        pltpu.make_async_copy(exb0, sh_den.at[dsts0], sx0).wait()
        pltpu.make_async_copy(wvb0, sh_num.at[dsts0], sw0).wait()
        pltpu.make_async_copy(exb1, sh_den.at[dsts1], sx1).wait()
        pltpu.make_async_copy(wvb1, sh_num.at[dsts1], sw1).wait()
        plsc.subcore_barrier()
        pltpu.sync_copy(sh_den.at[pl.ds(r0, rows)], out.at[c, 0, pl.ds(r0, rows)])
        pltpu.sync_copy(sh_num.at[pl.ds(r0, rows)], out.at[c, 1, pl.ds(r0, rows)])
        plsc.subcore_barrier()


def _edge_stage(kv, q, srcp, dstp, zer):
    meta = []
    ins = []
    for (s, d) in EDGE_TYPES:
        en = s + "__" + d
        ins.append(kv[en])
        meta.append((NODE_TYPES.index(d), N_NODES[d], N_PAD[d]))
    for t in NODE_TYPES:
        ins.append(q[t])
    for (s, d) in EDGE_TYPES:
        en = s + "__" + d
        ins += [srcp[en], dstp[en]]
    ins.append(zer)
    out_type = tuple(jax.ShapeDtypeStruct((2, 2, N_PAD[d], DP), jnp.float32) for (s, d) in EDGE_TYPES)
    mesh = plsc.VectorSubcoreMesh(core_axis_name="c", subcore_axis_name="s")
    f = pl.kernel(
        functools.partial(_edge_body, meta),
        out_type=out_type,
        mesh=mesh,
        scratch_types=[pltpu.VMEM((CHUNK,), jnp.int32)] * 8
        + [pltpu.VMEM((CHUNK, 2 * DP), jnp.float32)] * 2
        + [pltpu.VMEM((CHUNK, DP), jnp.float32)] * 6
        + [pltpu.VMEM_SHARED((N_PAD["paper"], DP), jnp.float32)] * 2
        + [pltpu.SemaphoreType.DMA] * 8,
    )
    return f(*ins)


def _post_body(n_en, slab0, *refs):
    slabs = (slab0,) + refs[:n_en - 1]
    wa_ref, ba_ref, out_ref = refs[n_en - 1:]
    agg = jnp.zeros_like(slabs[0][0, 0])
    for sl_ref in slabs:
        sl = sl_ref[...]
        den = sl[0, 0] + sl[1, 0]
        num = sl[0, 1] + sl[1, 1]
        agg = agg + num / (den + 1e-16)
    g = 0.5 * agg * (1.0 + jnp.tanh(0.7978845608028654 * (agg + 0.044715 * agg * agg * agg)))
    out_ref[...] = jnp.dot(g, wa_ref[...], preferred_element_type=jnp.float32) + ba_ref[...]


def _post(slabs, wa, ba, n):
    n_en = len(slabs)
    dout = wa.shape[1]
    R = 1000
    grid = n // R
    return pl.pallas_call(
        functools.partial(_post_body, n_en),
        out_shape=jax.ShapeDtypeStruct((n, dout), jnp.float32),
        grid=(grid,),
        in_specs=[pl.BlockSpec((2, 2, R, DP), lambda i: (0, 0, i, 0))] * n_en
        + [pl.BlockSpec((DP, dout), lambda i: (0, 0)),
           pl.BlockSpec((1, dout), lambda i: (0, 0))],
        out_specs=pl.BlockSpec((R, dout), lambda i: (i, 0)),
    )(*slabs, wa, ba.reshape(1, dout))


def _final_body(h_ref, w_ref, b_ref, o_ref):
    logits = jnp.dot(h_ref[...], w_ref[...], preferred_element_type=jnp.float32) + b_ref[...]
    m = jnp.max(logits, axis=1, keepdims=True)
    e = jnp.exp(logits - m)
    o_ref[...] = e / jnp.sum(e, axis=1, keepdims=True)


def _final(h, lin_W, lin_b):
    n = h.shape[0]
    return pl.pallas_call(
        _final_body,
        out_shape=jax.ShapeDtypeStruct((n, OUT_DIM), jnp.float32),
        grid=(10,),
        in_specs=[
            pl.BlockSpec((n // 10, 64), lambda i: (i, 0)),
            pl.BlockSpec((64, OUT_DIM), lambda i: (0, 0)),
            pl.BlockSpec((1, OUT_DIM), lambda i: (0, 0)),
        ],
        out_specs=pl.BlockSpec((n // 10, OUT_DIM), lambda i: (i, 0)),
    )(h, lin_W, lin_b.reshape(1, OUT_DIM))


def _layer(xd, srcp, dstp, zer, p, L):
    din, dout = DIMS[L]
    dh = dout // H
    scale = 1.0 / np.sqrt(dh)
    src_of = {t: [en for en in EDGE_TYPES if en[0] == t] for t in NODE_TYPES}
    dst_of = {t: [en for en in EDGE_TYPES if en[1] == t] for t in NODE_TYPES}
    q = {}
    kv = {}
    wa_pad = {}
    for t in NODE_TYPES:
        arel_s = []
        mrel = []
        for (s, d) in src_of[t]:
            en = s + "__" + d
            arel_s.append(p["c%d_arel_%s" % (L, en)]
                          * (p["c%d_prel_%s" % (L, en)] * scale)[:, None, None])
            mrel.append(p["c%d_mrel_%s" % (L, en)])
        pr = _prep(L, t, p, arel_s, mrel)
        wf, bf = pr[0], pr[1]
        if dh == 8:
            wa_pad[t] = pr[2]
        outs = _proj(xd[t], wf, bf, len(src_of[t]))
        q[t] = outs[0]
        for i, (s, d) in enumerate(src_of[t]):
            kv[s + "__" + d] = outs[1 + i]
    slabs = _edge_stage(kv, q, srcp, dstp, zer)
    slab_of = {s + "__" + d: slabs[i] for i, (s, d) in enumerate(EDGE_TYPES)}
    out = {}
    for t in NODE_TYPES:
        sl = [slab_of[s + "__" + d] for (s, d) in dst_of[t]]
        wa = wa_pad[t] if dh == 8 else p["c%d_Wa_%s" % (L, t)]
        out[t] = _post(sl, wa, p["c%d_ba_%s" % (L, t)], N_NODES[t])
    return out


def kernel(x_paper, x_author, x_term, edge_index_paper__author, edge_index_author__paper, edge_index_paper__term, edge_index_term__paper, c1_Wk_paper, c1_Wq_paper, c1_Wv_paper, c1_bk_paper, c1_bq_paper, c1_bv_paper, c1_Wa_paper, c1_ba_paper, c1_Wk_author, c1_Wq_author, c1_Wv_author, c1_bk_author, c1_bq_author, c1_bv_author, c1_Wa_author, c1_ba_author, c1_Wk_term, c1_Wq_term, c1_Wv_term, c1_bk_term, c1_bq_term, c1_bv_term, c1_Wa_term, c1_ba_term, c1_arel_paper__author, c1_mrel_paper__author, c1_prel_paper__author, c1_arel_author__paper, c1_mrel_author__paper, c1_prel_author__paper, c1_arel_paper__term, c1_mrel_paper__term, c1_prel_paper__term, c1_arel_term__paper, c1_mrel_term__paper, c1_prel_term__paper, c2_Wk_paper, c2_Wq_paper, c2_Wv_paper, c2_bk_paper, c2_bq_paper, c2_bv_paper, c2_Wa_paper, c2_ba_paper, c2_Wk_author, c2_Wq_author, c2_Wv_author, c2_bk_author, c2_bq_author, c2_bv_author, c2_Wa_author, c2_ba_author, c2_Wk_term, c2_Wq_term, c2_Wv_term, c2_bk_term, c2_bq_term, c2_bv_term, c2_Wa_term, c2_ba_term, c2_arel_paper__author, c2_mrel_paper__author, c2_prel_paper__author, c2_arel_author__paper, c2_mrel_author__paper, c2_prel_author__paper, c2_arel_paper__term, c2_mrel_paper__term, c2_prel_paper__term, c2_arel_term__paper, c2_mrel_term__paper, c2_prel_term__paper, lin_W, lin_b):
    kwargs = dict(locals())
    p = {kk: vv for kk, vv in kwargs.items() if not kk.startswith("edge_index_")}
    srcp = {}
    dstp = {}
    for (s, d) in EDGE_TYPES:
        en = s + "__" + d
        ei = kwargs["edge_index_" + en]
        pad = E_PAD - N_EDGES
        srcp[en] = jnp.concatenate([ei[0], jnp.zeros((pad,), ei.dtype)])
        dstp[en] = jnp.concatenate([ei[1], jnp.full((pad,), N_NODES[d], ei.dtype)])
    zer = jnp.zeros((N_PAD["paper"], DP), jnp.float32)
    xd = {t: p["x_" + t] for t in NODE_TYPES}
    h1 = _layer(xd, srcp, dstp, zer, p, 1)
    h2 = _layer(h1, srcp, dstp, zer, p, 2)
    hcat = jnp.vstack([h2[t] for t in NODE_TYPES])
    return _final(hcat, lin_W, lin_b)


# speedup vs baseline: 45.4267x; 1.1278x over previous
import functools

import jax
import jax.numpy as jnp
import numpy as np
from jax import lax
from jax.experimental import pallas as pl
from jax.experimental.pallas import tpu as pltpu
from jax.experimental.pallas import tpu_sc as plsc

NODE_TYPES = ("paper", "author", "term")
N_NODES = {"paper": 4000, "author": 3000, "term": 3000}
N_PAD = {"paper": 4096, "author": 3072, "term": 3072}
EDGE_TYPES = (("paper", "author"), ("author", "paper"), ("paper", "term"), ("term", "paper"))
N_EDGES = 40000
E_PAD = 40960
H = 8
DIMS = {1: (2048, 128), 2: (128, 64)}
OUT_DIM = 16
DP = 128
NW = 32
CHUNK = 32
CHUNKS_PER_W = E_PAD // (NW * CHUNK)


def _prep_body(dh, n_src, *refs):
    wk, wq, wv, bk, bq, bv = refs[:6]
    rels = refs[6:6 + 2 * n_src]
    if dh == 8:
        wa = refs[6 + 2 * n_src]
        wf_ref, bf_ref, wap_ref = refs[7 + 2 * n_src:]
    else:
        wf_ref, bf_ref = refs[6 + 2 * n_src:]

    def padcols(m):
        if dh == 16:
            return m
        return jnp.concatenate([m, jnp.zeros((m.shape[0], 16 - dh), jnp.float32)], axis=1)

    def heads(mat, rel=None):
        cols = []
        for h in range(H):
            blk = mat[:, h * dh:(h + 1) * dh]
            if rel is not None:
                blk = jnp.dot(blk, rel[h], preferred_element_type=jnp.float32)
            cols.append(padcols(blk))
        return jnp.concatenate(cols, axis=1)

    wcols = [heads(wq[...])]
    bcols = [heads(bq[...])]
    for i in range(n_src):
        arel_s = rels[2 * i][...]
        mrel = rels[2 * i + 1][...]
        wcols += [heads(wk[...], arel_s), heads(wv[...], mrel)]
        bcols += [heads(bk[...], arel_s), heads(bv[...], mrel)]
    wf_ref[...] = jnp.concatenate(wcols, axis=1)
    bf_ref[...] = jnp.concatenate(bcols, axis=1)
    if dh == 8:
        wav = wa[...]
        blocks = []
        for h in range(H):
            blocks.append(wav[h * 8:(h + 1) * 8, :])
            blocks.append(jnp.zeros((8, wav.shape[1]), jnp.float32))
        wap_ref[...] = jnp.concatenate(blocks, axis=0)


def _prep(L, t, p, arel_s, mrel):
    din, dout = DIMS[L]
    dh = dout // H
    n_src = len(arel_s)
    K = DP * (1 + 2 * n_src)
    ins = [p["c%d_Wk_%s" % (L, t)], p["c%d_Wq_%s" % (L, t)], p["c%d_Wv_%s" % (L, t)],
           p["c%d_bk_%s" % (L, t)].reshape(1, dout), p["c%d_bq_%s" % (L, t)].reshape(1, dout),
           p["c%d_bv_%s" % (L, t)].reshape(1, dout)]
    for a, m in zip(arel_s, mrel):
        ins += [a, m]
    outs = [jax.ShapeDtypeStruct((din, K), jnp.float32), jax.ShapeDtypeStruct((1, K), jnp.float32)]
    if dh == 8:
        ins.append(p["c%d_Wa_%s" % (L, t)])
        outs.append(jax.ShapeDtypeStruct((DP, dout), jnp.float32))
    return pl.pallas_call(
        functools.partial(_prep_body, dh, n_src),
        out_shape=tuple(outs),
    )(*ins)


def _proj_body(n_kv, x_ref, wf_ref, bf_ref, q_ref, *kv_refs):
    y = jnp.dot(x_ref[...], wf_ref[...], preferred_element_type=jnp.float32) + bf_ref[...]
    q_ref[...] = y[:, :DP]
    for i in range(n_kv):
        kv_refs[i][...] = y[:, DP + 2 * DP * i:DP + 2 * DP * (i + 1)]


def _proj(x, wf, bf, n_kv):
    n, din = x.shape
    K = wf.shape[1]
    R = 1000
    grid = n // R
    return pl.pallas_call(
        functools.partial(_proj_body, n_kv),
        out_shape=tuple([jax.ShapeDtypeStruct((n, DP), jnp.float32)]
                        + [jax.ShapeDtypeStruct((n, 2 * DP), jnp.float32)] * n_kv),
        grid=(grid,),
        in_specs=[
            pl.BlockSpec((R, din), lambda i: (i, 0)),
            pl.BlockSpec((din, K), lambda i: (0, 0)),
            pl.BlockSpec((1, K), lambda i: (0, 0)),
        ],
        out_specs=tuple([pl.BlockSpec((R, DP), lambda i: (i, 0))]
                        + [pl.BlockSpec((R, 2 * DP), lambda i: (i, 0))] * n_kv),
    )(x, wf, bf)


def _edge_body(meta, *refs):
    kvs = refs[0:4]
    qs = refs[4:7]
    idx = refs[7:15]
    zer = refs[15]
    outs = refs[16:20]
    (dstg0, dstg1, dsts0, dsts1, srcall, dstall, kvb0, kvb1, qb0, qb1,
     exb0, exb1, wvb0, wvb1, sh_den, sh_num,
     skv0, skv1, sq0, sq1, sx0, sx1, sw0, sw1, sz0, sz1) = refs[20:]
    c = lax.axis_index("c")
    s = lax.axis_index("s")
    w = s * 2 + c
    lane = lax.broadcasted_iota(jnp.int32, (16,), 0)
    gdn = lax.GatherDimensionNumbers(offset_dims=(), collapsed_slice_dims=(0,), start_index_map=(0,))
    shuf_idx = [(lane ^ off).reshape(16, 1) for off in (8, 4, 2, 1)]

    def allsum(x):
        for idx in shuf_idx:
            x = x + lax.gather(x, idx, gdn, (1,),
                               mode=lax.GatherScatterMode.PROMISE_IN_BOUNDS)
        return x
    for ti, (qi, nd, ndp) in enumerate(meta):
        kv = kvs[ti]
        q = qs[qi]
        se, de = idx[2 * ti], idx[2 * ti + 1]
        out = outs[ti]
        rows = ndp // 16
        r0 = s * rows
        pltpu.async_copy(zer.at[pl.ds(r0, rows)], sh_den.at[pl.ds(r0, rows)], sz0)
        pltpu.async_copy(zer.at[pl.ds(r0, rows)], sh_num.at[pl.ds(r0, rows)], sz1)
        wspan = CHUNKS_PER_W * CHUNK
        pltpu.sync_copy(se.at[pl.ds(w * wspan, wspan)], srcall)
        pltpu.sync_copy(de.at[pl.ds(w * wspan, wspan)], dstall)
        pltpu.make_async_copy(zer.at[pl.ds(r0, rows)], sh_den.at[pl.ds(r0, rows)], sz0).wait()
        pltpu.make_async_copy(zer.at[pl.ds(r0, rows)], sh_num.at[pl.ds(r0, rows)], sz1).wait()
        plsc.subcore_barrier()

        def fire(g, dg, kb, qbf, sk, sq):
            off = g * CHUNK
            for i in range(CHUNK // 16):
                dg[pl.ds(i * 16, 16)] = jnp.minimum(dstall[pl.ds(off + i * 16, 16)], nd - 1)
            pltpu.async_copy(kv.at[srcall.at[pl.ds(off, CHUNK)]], kb, sk)
            pltpu.async_copy(q.at[dg], qbf, sq)

        def drain_compute_scatter(p, g, dg, kb, qbf, sk, sq, xb, wb, ds_, sx, sw):
            off = g * CHUNK
            pltpu.make_async_copy(kv.at[srcall.at[pl.ds(off, CHUNK)]], kb, sk).wait()
            pltpu.make_async_copy(q.at[dg], qbf, sq).wait()

            @pl.when(p > 0)
            def _():
                pltpu.make_async_copy(xb, sh_den.at[ds_], sx).wait()
                pltpu.make_async_copy(wb, sh_num.at[ds_], sw).wait()

            for i in range(CHUNK // 16):
                ds_[pl.ds(i * 16, 16)] = dstall[pl.ds(off + i * 16, 16)]

            @plsc.parallel_loop(0, CHUNK, 1, unroll=2)
            def edge(e):
                for h in range(H):
                    kvv = kb[e, pl.ds(h * 16, 16)]
                    qv = qbf[e, pl.ds(h * 16, 16)]
                    exv = jnp.exp(allsum(qv * kvv))
                    xb[e, pl.ds(h * 16, 16)] = exv
                    wb[e, pl.ds(h * 16, 16)] = exv * kb[e, pl.ds(DP + h * 16, 16)]

            pltpu.async_copy(xb, sh_den.at[ds_], sx, add=True)
            pltpu.async_copy(wb, sh_num.at[ds_], sw, add=True)

        bufs0 = (dstg0, kvb0, qb0, skv0, sq0)
        bufs1 = (dstg1, kvb1, qb1, skv1, sq1)
        sc0 = (exb0, wvb0, dsts0, sx0, sw0)
        sc1 = (exb1, wvb1, dsts1, sx1, sw1)
        pairs = CHUNKS_PER_W // 2
        fire(0, *bufs0)

        def pair(p, _):
            fire(2 * p + 1, *bufs1)
            drain_compute_scatter(p, 2 * p, *bufs0, *sc0)

            @pl.when(p < pairs - 1)
            def _():
                fire(2 * p + 2, *bufs0)

            drain_compute_scatter(p, 2 * p + 1, *bufs1, *sc1)
            return 0

        lax.fori_loop(0, pairs, pair, 0)
        pltpu.make_async_copy(exb0, sh_den.at[dsts0], sx0).wait()
        pltpu.make_async_copy(wvb0, sh_num.at[dsts0], sw0).wait()
        pltpu.make_async_copy(exb1, sh_den.at[dsts1], sx1).wait()
        pltpu.make_async_copy(wvb1, sh_num.at[dsts1], sw1).wait()
        plsc.subcore_barrier()
        pltpu.sync_copy(sh_den.at[pl.ds(r0, rows)], out.at[c, 0, pl.ds(r0, rows)])
        pltpu.sync_copy(sh_num.at[pl.ds(r0, rows)], out.at[c, 1, pl.ds(r0, rows)])
        plsc.subcore_barrier()


def _edge_stage(kv, q, srcp, dstp, zer):
    meta = []
    ins = []
    for (s, d) in EDGE_TYPES:
        en = s + "__" + d
        ins.append(kv[en])
        meta.append((NODE_TYPES.index(d), N_NODES[d], N_PAD[d]))
    for t in NODE_TYPES:
        ins.append(q[t])
    for (s, d) in EDGE_TYPES:
        en = s + "__" + d
        ins += [srcp[en], dstp[en]]
    ins.append(zer)
    out_type = tuple(jax.ShapeDtypeStruct((2, 2, N_PAD[d], DP), jnp.float32) for (s, d) in EDGE_TYPES)
    mesh = plsc.VectorSubcoreMesh(core_axis_name="c", subcore_axis_name="s")
    f = pl.kernel(
        functools.partial(_edge_body, meta),
        out_type=out_type,
        mesh=mesh,
        scratch_types=[pltpu.VMEM((CHUNK,), jnp.int32)] * 4
        + [pltpu.VMEM((CHUNKS_PER_W * CHUNK,), jnp.int32)] * 2
        + [pltpu.VMEM((CHUNK, 2 * DP), jnp.float32)] * 2
        + [pltpu.VMEM((CHUNK, DP), jnp.float32)] * 6
        + [pltpu.VMEM_SHARED((N_PAD["paper"], DP), jnp.float32)] * 2
        + [pltpu.SemaphoreType.DMA] * 10,
    )
    return f(*ins)


def _post_body(n_en, slab0, *refs):
    slabs = (slab0,) + refs[:n_en - 1]
    wa_ref, ba_ref, out_ref = refs[n_en - 1:]
    agg = jnp.zeros_like(slabs[0][0, 0])
    for sl_ref in slabs:
        sl = sl_ref[...]
        den = sl[0, 0] + sl[1, 0]
        num = sl[0, 1] + sl[1, 1]
        agg = agg + num / (den + 1e-16)
    g = 0.5 * agg * (1.0 + jnp.tanh(0.7978845608028654 * (agg + 0.044715 * agg * agg * agg)))
    out_ref[...] = jnp.dot(g, wa_ref[...], preferred_element_type=jnp.float32) + ba_ref[...]


def _post(slabs, wa, ba, n):
    n_en = len(slabs)
    dout = wa.shape[1]
    R = 1000
    grid = n // R
    return pl.pallas_call(
        functools.partial(_post_body, n_en),
        out_shape=jax.ShapeDtypeStruct((n, dout), jnp.float32),
        grid=(grid,),
        in_specs=[pl.BlockSpec((2, 2, R, DP), lambda i: (0, 0, i, 0))] * n_en
        + [pl.BlockSpec((DP, dout), lambda i: (0, 0)),
           pl.BlockSpec((1, dout), lambda i: (0, 0))],
        out_specs=pl.BlockSpec((R, dout), lambda i: (i, 0)),
    )(*slabs, wa, ba.reshape(1, dout))


def _final_body(h_ref, w_ref, b_ref, o_ref):
    logits = jnp.dot(h_ref[...], w_ref[...], preferred_element_type=jnp.float32) + b_ref[...]
    m = jnp.max(logits, axis=1, keepdims=True)
    e = jnp.exp(logits - m)
    o_ref[...] = e / jnp.sum(e, axis=1, keepdims=True)


def _final(h, lin_W, lin_b):
    n = h.shape[0]
    return pl.pallas_call(
        _final_body,
        out_shape=jax.ShapeDtypeStruct((n, OUT_DIM), jnp.float32),
        grid=(10,),
        in_specs=[
            pl.BlockSpec((n // 10, 64), lambda i: (i, 0)),
            pl.BlockSpec((64, OUT_DIM), lambda i: (0, 0)),
            pl.BlockSpec((1, OUT_DIM), lambda i: (0, 0)),
        ],
        out_specs=pl.BlockSpec((n // 10, OUT_DIM), lambda i: (i, 0)),
    )(h, lin_W, lin_b.reshape(1, OUT_DIM))


def _layer(xd, srcp, dstp, zer, p, L):
    din, dout = DIMS[L]
    dh = dout // H
    scale = 1.0 / np.sqrt(dh)
    src_of = {t: [en for en in EDGE_TYPES if en[0] == t] for t in NODE_TYPES}
    dst_of = {t: [en for en in EDGE_TYPES if en[1] == t] for t in NODE_TYPES}
    q = {}
    kv = {}
    wa_pad = {}
    for t in NODE_TYPES:
        arel_s = []
        mrel = []
        for (s, d) in src_of[t]:
            en = s + "__" + d
            arel_s.append(p["c%d_arel_%s" % (L, en)]
                          * (p["c%d_prel_%s" % (L, en)] * scale)[:, None, None])
            mrel.append(p["c%d_mrel_%s" % (L, en)])
        pr = _prep(L, t, p, arel_s, mrel)
        wf, bf = pr[0], pr[1]
        if dh == 8:
            wa_pad[t] = pr[2]
        outs = _proj(xd[t], wf, bf, len(src_of[t]))
        q[t] = outs[0]
        for i, (s, d) in enumerate(src_of[t]):
            kv[s + "__" + d] = outs[1 + i]
    slabs = _edge_stage(kv, q, srcp, dstp, zer)
    slab_of = {s + "__" + d: slabs[i] for i, (s, d) in enumerate(EDGE_TYPES)}
    out = {}
    for t in NODE_TYPES:
        sl = [slab_of[s + "__" + d] for (s, d) in dst_of[t]]
        wa = wa_pad[t] if dh == 8 else p["c%d_Wa_%s" % (L, t)]
        out[t] = _post(sl, wa, p["c%d_ba_%s" % (L, t)], N_NODES[t])
    return out


def kernel(x_paper, x_author, x_term, edge_index_paper__author, edge_index_author__paper, edge_index_paper__term, edge_index_term__paper, c1_Wk_paper, c1_Wq_paper, c1_Wv_paper, c1_bk_paper, c1_bq_paper, c1_bv_paper, c1_Wa_paper, c1_ba_paper, c1_Wk_author, c1_Wq_author, c1_Wv_author, c1_bk_author, c1_bq_author, c1_bv_author, c1_Wa_author, c1_ba_author, c1_Wk_term, c1_Wq_term, c1_Wv_term, c1_bk_term, c1_bq_term, c1_bv_term, c1_Wa_term, c1_ba_term, c1_arel_paper__author, c1_mrel_paper__author, c1_prel_paper__author, c1_arel_author__paper, c1_mrel_author__paper, c1_prel_author__paper, c1_arel_paper__term, c1_mrel_paper__term, c1_prel_paper__term, c1_arel_term__paper, c1_mrel_term__paper, c1_prel_term__paper, c2_Wk_paper, c2_Wq_paper, c2_Wv_paper, c2_bk_paper, c2_bq_paper, c2_bv_paper, c2_Wa_paper, c2_ba_paper, c2_Wk_author, c2_Wq_author, c2_Wv_author, c2_bk_author, c2_bq_author, c2_bv_author, c2_Wa_author, c2_ba_author, c2_Wk_term, c2_Wq_term, c2_Wv_term, c2_bk_term, c2_bq_term, c2_bv_term, c2_Wa_term, c2_ba_term, c2_arel_paper__author, c2_mrel_paper__author, c2_prel_paper__author, c2_arel_author__paper, c2_mrel_author__paper, c2_prel_author__paper, c2_arel_paper__term, c2_mrel_paper__term, c2_prel_paper__term, c2_arel_term__paper, c2_mrel_term__paper, c2_prel_term__paper, lin_W, lin_b):
    kwargs = dict(locals())
    p = {kk: vv for kk, vv in kwargs.items() if not kk.startswith("edge_index_")}
    srcp = {}
    dstp = {}
    for (s, d) in EDGE_TYPES:
        en = s + "__" + d
        ei = kwargs["edge_index_" + en]
        pad = E_PAD - N_EDGES
        srcp[en] = jnp.concatenate([ei[0], jnp.zeros((pad,), ei.dtype)])
        dstp[en] = jnp.concatenate([ei[1], jnp.full((pad,), N_NODES[d], ei.dtype)])
    zer = jnp.zeros((N_PAD["paper"], DP), jnp.float32)
    xd = {t: p["x_" + t] for t in NODE_TYPES}
    h1 = _layer(xd, srcp, dstp, zer, p, 1)
    h2 = _layer(h1, srcp, dstp, zer, p, 2)
    hcat = jnp.vstack([h2[t] for t in NODE_TYPES])
    return _final(hcat, lin_W, lin_b)


# speedup vs baseline: 45.4713x; 1.0010x over previous
import functools

import jax
import jax.numpy as jnp
import numpy as np
from jax import lax
from jax.experimental import pallas as pl
from jax.experimental.pallas import tpu as pltpu
from jax.experimental.pallas import tpu_sc as plsc

NODE_TYPES = ("paper", "author", "term")
N_NODES = {"paper": 4000, "author": 3000, "term": 3000}
N_PAD = {"paper": 4096, "author": 3072, "term": 3072}
EDGE_TYPES = (("paper", "author"), ("author", "paper"), ("paper", "term"), ("term", "paper"))
N_EDGES = 40000
E_PAD = 40960
H = 8
DIMS = {1: (2048, 128), 2: (128, 64)}
OUT_DIM = 16
DP = 128
NW = 32
CHUNK = 32
CHUNKS_PER_W = E_PAD // (NW * CHUNK)


def _prep_body(dh, n_src, *refs):
    wk, wq, wv, bk, bq, bv = refs[:6]
    rels = refs[6:6 + 2 * n_src]
    if dh == 8:
        wa = refs[6 + 2 * n_src]
        wf_ref, bf_ref, wap_ref = refs[7 + 2 * n_src:]
    else:
        wf_ref, bf_ref = refs[6 + 2 * n_src:]

    def padcols(m):
        if dh == 16:
            return m
        return jnp.concatenate([m, jnp.zeros((m.shape[0], 16 - dh), jnp.float32)], axis=1)

    def heads(mat, rel=None):
        cols = []
        for h in range(H):
            blk = mat[:, h * dh:(h + 1) * dh]
            if rel is not None:
                blk = jnp.dot(blk, rel[h], preferred_element_type=jnp.float32)
            cols.append(padcols(blk))
        return jnp.concatenate(cols, axis=1)

    wcols = [heads(wq[...])]
    bcols = [heads(bq[...])]
    for i in range(n_src):
        arel_s = rels[2 * i][...]
        mrel = rels[2 * i + 1][...]
        wcols += [heads(wk[...], arel_s), heads(wv[...], mrel)]
        bcols += [heads(bk[...], arel_s), heads(bv[...], mrel)]
    wf_ref[...] = jnp.concatenate(wcols, axis=1)
    bf_ref[...] = jnp.concatenate(bcols, axis=1)
    if dh == 8:
        wav = wa[...]
        blocks = []
        for h in range(H):
            blocks.append(wav[h * 8:(h + 1) * 8, :])
            blocks.append(jnp.zeros((8, wav.shape[1]), jnp.float32))
        wap_ref[...] = jnp.concatenate(blocks, axis=0)


def _prep(L, t, p, arel_s, mrel):
    din, dout = DIMS[L]
    dh = dout // H
    n_src = len(arel_s)
    K = DP * (1 + 2 * n_src)
    ins = [p["c%d_Wk_%s" % (L, t)], p["c%d_Wq_%s" % (L, t)], p["c%d_Wv_%s" % (L, t)],
           p["c%d_bk_%s" % (L, t)].reshape(1, dout), p["c%d_bq_%s" % (L, t)].reshape(1, dout),
           p["c%d_bv_%s" % (L, t)].reshape(1, dout)]
    for a, m in zip(arel_s, mrel):
        ins += [a, m]
    outs = [jax.ShapeDtypeStruct((din, K), jnp.float32), jax.ShapeDtypeStruct((1, K), jnp.float32)]
    if dh == 8:
        ins.append(p["c%d_Wa_%s" % (L, t)])
        outs.append(jax.ShapeDtypeStruct((DP, dout), jnp.float32))
    return pl.pallas_call(
        functools.partial(_prep_body, dh, n_src),
        out_shape=tuple(outs),
    )(*ins)


def _proj_body(n_kv, x_ref, wf_ref, bf_ref, q_ref, *kv_refs):
    y = jax.lax.dot_general(
        x_ref[...], wf_ref[...], (((1,), (0,)), ((), ())),
        precision=jax.lax.Precision.DEFAULT,
        preferred_element_type=jnp.float32) + bf_ref[...]
    q_ref[...] = y[:, :DP]
    for i in range(n_kv):
        kv_refs[i][...] = y[:, DP + 2 * DP * i:DP + 2 * DP * (i + 1)]


def _proj(x, wf, bf, n_kv):
    n, din = x.shape
    K = wf.shape[1]
    R = 1000
    grid = n // R
    return pl.pallas_call(
        functools.partial(_proj_body, n_kv),
        out_shape=tuple([jax.ShapeDtypeStruct((n, DP), jnp.float32)]
                        + [jax.ShapeDtypeStruct((n, 2 * DP), jnp.float32)] * n_kv),
        grid=(grid,),
        in_specs=[
            pl.BlockSpec((R, din), lambda i: (i, 0)),
            pl.BlockSpec((din, K), lambda i: (0, 0)),
            pl.BlockSpec((1, K), lambda i: (0, 0)),
        ],
        out_specs=tuple([pl.BlockSpec((R, DP), lambda i: (i, 0))]
                        + [pl.BlockSpec((R, 2 * DP), lambda i: (i, 0))] * n_kv),
    )(x, wf, bf)


def _edge_body(meta, *refs):
    kvs = refs[0:4]
    qs = refs[4:7]
    idx = refs[7:15]
    zer = refs[15]
    outs = refs[16:20]
    (dstg0, dstg1, dsts0, dsts1, srcall, dstall, kvb0, kvb1, qb0, qb1,
     exb0, exb1, wvb0, wvb1, sh_den, sh_num,
     skv0, skv1, sq0, sq1, sx0, sx1, sw0, sw1, sz0, sz1) = refs[20:]
    c = lax.axis_index("c")
    s = lax.axis_index("s")
    w = s * 2 + c
    lane = lax.broadcasted_iota(jnp.int32, (16,), 0)
    gdn = lax.GatherDimensionNumbers(offset_dims=(), collapsed_slice_dims=(0,), start_index_map=(0,))
    shuf_idx = [(lane ^ off).reshape(16, 1) for off in (8, 4, 2, 1)]

    def allsum(x):
        for idx in shuf_idx:
            x = x + lax.gather(x, idx, gdn, (1,),
                               mode=lax.GatherScatterMode.PROMISE_IN_BOUNDS)
        return x
    for ti, (qi, nd, ndp) in enumerate(meta):
        kv = kvs[ti]
        q = qs[qi]
        se, de = idx[2 * ti], idx[2 * ti + 1]
        out = outs[ti]
        rows = ndp // 16
        r0 = s * rows
        pltpu.async_copy(zer.at[pl.ds(r0, rows)], sh_den.at[pl.ds(r0, rows)], sz0)
        pltpu.async_copy(zer.at[pl.ds(r0, rows)], sh_num.at[pl.ds(r0, rows)], sz1)
        wspan = CHUNKS_PER_W * CHUNK
        pltpu.sync_copy(se.at[pl.ds(w * wspan, wspan)], srcall)
        pltpu.sync_copy(de.at[pl.ds(w * wspan, wspan)], dstall)
        pltpu.make_async_copy(zer.at[pl.ds(r0, rows)], sh_den.at[pl.ds(r0, rows)], sz0).wait()
        pltpu.make_async_copy(zer.at[pl.ds(r0, rows)], sh_num.at[pl.ds(r0, rows)], sz1).wait()
        plsc.subcore_barrier()

        def fire(g, dg, kb, qbf, sk, sq):
            off = g * CHUNK
            for i in range(CHUNK // 16):
                dg[pl.ds(i * 16, 16)] = jnp.minimum(dstall[pl.ds(off + i * 16, 16)], nd - 1)
            pltpu.async_copy(kv.at[srcall.at[pl.ds(off, CHUNK)]], kb, sk)
            pltpu.async_copy(q.at[dg], qbf, sq)

        def drain_compute_scatter(p, g, dg, kb, qbf, sk, sq, xb, wb, ds_, sx, sw):
            off = g * CHUNK
            pltpu.make_async_copy(kv.at[srcall.at[pl.ds(off, CHUNK)]], kb, sk).wait()
            pltpu.make_async_copy(q.at[dg], qbf, sq).wait()

            @pl.when(p > 0)
            def _():
                pltpu.make_async_copy(xb, sh_den.at[ds_], sx).wait()
                pltpu.make_async_copy(wb, sh_num.at[ds_], sw).wait()

            for i in range(CHUNK // 16):
                ds_[pl.ds(i * 16, 16)] = dstall[pl.ds(off + i * 16, 16)]

            @plsc.parallel_loop(0, CHUNK, 1, unroll=2)
            def edge(e):
                for h in range(H):
                    kvv = kb[e, pl.ds(h * 16, 16)]
                    qv = qbf[e, pl.ds(h * 16, 16)]
                    exv = jnp.exp(allsum(qv * kvv))
                    xb[e, pl.ds(h * 16, 16)] = exv
                    wb[e, pl.ds(h * 16, 16)] = exv * kb[e, pl.ds(DP + h * 16, 16)]

            pltpu.async_copy(xb, sh_den.at[ds_], sx, add=True)
            pltpu.async_copy(wb, sh_num.at[ds_], sw, add=True)

        bufs0 = (dstg0, kvb0, qb0, skv0, sq0)
        bufs1 = (dstg1, kvb1, qb1, skv1, sq1)
        sc0 = (exb0, wvb0, dsts0, sx0, sw0)
        sc1 = (exb1, wvb1, dsts1, sx1, sw1)
        pairs = CHUNKS_PER_W // 2
        fire(0, *bufs0)

        def pair(p, _):
            fire(2 * p + 1, *bufs1)
            drain_compute_scatter(p, 2 * p, *bufs0, *sc0)

            @pl.when(p < pairs - 1)
            def _():
                fire(2 * p + 2, *bufs0)

            drain_compute_scatter(p, 2 * p + 1, *bufs1, *sc1)
            return 0

        lax.fori_loop(0, pairs, pair, 0)
        pltpu.make_async_copy(exb0, sh_den.at[dsts0], sx0).wait()
        pltpu.make_async_copy(wvb0, sh_num.at[dsts0], sw0).wait()
        pltpu.make_async_copy(exb1, sh_den.at[dsts1], sx1).wait()
        pltpu.make_async_copy(wvb1, sh_num.at[dsts1], sw1).wait()
        plsc.subcore_barrier()
        pltpu.sync_copy(sh_den.at[pl.ds(r0, rows)], out.at[c, 0, pl.ds(r0, rows)])
        pltpu.sync_copy(sh_num.at[pl.ds(r0, rows)], out.at[c, 1, pl.ds(r0, rows)])
        plsc.subcore_barrier()


def _edge_stage(kv, q, srcp, dstp, zer):
    meta = []
    ins = []
    for (s, d) in EDGE_TYPES:
        en = s + "__" + d
        ins.append(kv[en])
        meta.append((NODE_TYPES.index(d), N_NODES[d], N_PAD[d]))
    for t in NODE_TYPES:
        ins.append(q[t])
    for (s, d) in EDGE_TYPES:
        en = s + "__" + d
        ins += [srcp[en], dstp[en]]
    ins.append(zer)
    out_type = tuple(jax.ShapeDtypeStruct((2, 2, N_PAD[d], DP), jnp.float32) for (s, d) in EDGE_TYPES)
    mesh = plsc.VectorSubcoreMesh(core_axis_name="c", subcore_axis_name="s")
    f = pl.kernel(
        functools.partial(_edge_body, meta),
        out_type=out_type,
        mesh=mesh,
        scratch_types=[pltpu.VMEM((CHUNK,), jnp.int32)] * 4
        + [pltpu.VMEM((CHUNKS_PER_W * CHUNK,), jnp.int32)] * 2
        + [pltpu.VMEM((CHUNK, 2 * DP), jnp.float32)] * 2
        + [pltpu.VMEM((CHUNK, DP), jnp.float32)] * 6
        + [pltpu.VMEM_SHARED((N_PAD["paper"], DP), jnp.float32)] * 2
        + [pltpu.SemaphoreType.DMA] * 10,
    )
    return f(*ins)


def _post_body(n_en, slab0, *refs):
    slabs = (slab0,) + refs[:n_en - 1]
    wa_ref, ba_ref, out_ref = refs[n_en - 1:]
    agg = jnp.zeros_like(slabs[0][0, 0])
    for sl_ref in slabs:
        sl = sl_ref[...]
        den = sl[0, 0] + sl[1, 0]
        num = sl[0, 1] + sl[1, 1]
        agg = agg + num / (den + 1e-16)
    g = 0.5 * agg * (1.0 + jnp.tanh(0.7978845608028654 * (agg + 0.044715 * agg * agg * agg)))
    out_ref[...] = jnp.dot(g, wa_ref[...], preferred_element_type=jnp.float32) + ba_ref[...]


def _post(slabs, wa, ba, n):
    n_en = len(slabs)
    dout = wa.shape[1]
    R = 1000
    grid = n // R
    return pl.pallas_call(
        functools.partial(_post_body, n_en),
        out_shape=jax.ShapeDtypeStruct((n, dout), jnp.float32),
        grid=(grid,),
        in_specs=[pl.BlockSpec((2, 2, R, DP), lambda i: (0, 0, i, 0))] * n_en
        + [pl.BlockSpec((DP, dout), lambda i: (0, 0)),
           pl.BlockSpec((1, dout), lambda i: (0, 0))],
        out_specs=pl.BlockSpec((R, dout), lambda i: (i, 0)),
    )(*slabs, wa, ba.reshape(1, dout))


def _final_body(h_ref, w_ref, b_ref, o_ref):
    logits = jnp.dot(h_ref[...], w_ref[...], preferred_element_type=jnp.float32) + b_ref[...]
    m = jnp.max(logits, axis=1, keepdims=True)
    e = jnp.exp(logits - m)
    o_ref[...] = e / jnp.sum(e, axis=1, keepdims=True)


def _final(h, lin_W, lin_b):
    n = h.shape[0]
    return pl.pallas_call(
        _final_body,
        out_shape=jax.ShapeDtypeStruct((n, OUT_DIM), jnp.float32),
        grid=(10,),
        in_specs=[
            pl.BlockSpec((n // 10, 64), lambda i: (i, 0)),
            pl.BlockSpec((64, OUT_DIM), lambda i: (0, 0)),
            pl.BlockSpec((1, OUT_DIM), lambda i: (0, 0)),
        ],
        out_specs=pl.BlockSpec((n // 10, OUT_DIM), lambda i: (i, 0)),
    )(h, lin_W, lin_b.reshape(1, OUT_DIM))


def _layer(xd, srcp, dstp, zer, p, L):
    din, dout = DIMS[L]
    dh = dout // H
    scale = 1.0 / np.sqrt(dh)
    src_of = {t: [en for en in EDGE_TYPES if en[0] == t] for t in NODE_TYPES}
    dst_of = {t: [en for en in EDGE_TYPES if en[1] == t] for t in NODE_TYPES}
    q = {}
    kv = {}
    wa_pad = {}
    for t in NODE_TYPES:
        arel_s = []
        mrel = []
        for (s, d) in src_of[t]:
            en = s + "__" + d
            arel_s.append(p["c%d_arel_%s" % (L, en)]
                          * (p["c%d_prel_%s" % (L, en)] * scale)[:, None, None])
            mrel.append(p["c%d_mrel_%s" % (L, en)])
        pr = _prep(L, t, p, arel_s, mrel)
        wf, bf = pr[0], pr[1]
        if dh == 8:
            wa_pad[t] = pr[2]
        outs = _proj(xd[t], wf, bf, len(src_of[t]))
        q[t] = outs[0]
        for i, (s, d) in enumerate(src_of[t]):
            kv[s + "__" + d] = outs[1 + i]
    slabs = _edge_stage(kv, q, srcp, dstp, zer)
    slab_of = {s + "__" + d: slabs[i] for i, (s, d) in enumerate(EDGE_TYPES)}
    out = {}
    for t in NODE_TYPES:
        sl = [slab_of[s + "__" + d] for (s, d) in dst_of[t]]
        wa = wa_pad[t] if dh == 8 else p["c%d_Wa_%s" % (L, t)]
        out[t] = _post(sl, wa, p["c%d_ba_%s" % (L, t)], N_NODES[t])
    return out


def kernel(x_paper, x_author, x_term, edge_index_paper__author, edge_index_author__paper, edge_index_paper__term, edge_index_term__paper, c1_Wk_paper, c1_Wq_paper, c1_Wv_paper, c1_bk_paper, c1_bq_paper, c1_bv_paper, c1_Wa_paper, c1_ba_paper, c1_Wk_author, c1_Wq_author, c1_Wv_author, c1_bk_author, c1_bq_author, c1_bv_author, c1_Wa_author, c1_ba_author, c1_Wk_term, c1_Wq_term, c1_Wv_term, c1_bk_term, c1_bq_term, c1_bv_term, c1_Wa_term, c1_ba_term, c1_arel_paper__author, c1_mrel_paper__author, c1_prel_paper__author, c1_arel_author__paper, c1_mrel_author__paper, c1_prel_author__paper, c1_arel_paper__term, c1_mrel_paper__term, c1_prel_paper__term, c1_arel_term__paper, c1_mrel_term__paper, c1_prel_term__paper, c2_Wk_paper, c2_Wq_paper, c2_Wv_paper, c2_bk_paper, c2_bq_paper, c2_bv_paper, c2_Wa_paper, c2_ba_paper, c2_Wk_author, c2_Wq_author, c2_Wv_author, c2_bk_author, c2_bq_author, c2_bv_author, c2_Wa_author, c2_ba_author, c2_Wk_term, c2_Wq_term, c2_Wv_term, c2_bk_term, c2_bq_term, c2_bv_term, c2_Wa_term, c2_ba_term, c2_arel_paper__author, c2_mrel_paper__author, c2_prel_paper__author, c2_arel_author__paper, c2_mrel_author__paper, c2_prel_author__paper, c2_arel_paper__term, c2_mrel_paper__term, c2_prel_paper__term, c2_arel_term__paper, c2_mrel_term__paper, c2_prel_term__paper, lin_W, lin_b):
    kwargs = dict(locals())
    p = {kk: vv for kk, vv in kwargs.items() if not kk.startswith("edge_index_")}
    srcp = {}
    dstp = {}
    for (s, d) in EDGE_TYPES:
        en = s + "__" + d
        ei = kwargs["edge_index_" + en]
        pad = E_PAD - N_EDGES
        srcp[en] = jnp.concatenate([ei[0], jnp.zeros((pad,), ei.dtype)])
        dstp[en] = jnp.concatenate([ei[1], jnp.full((pad,), N_NODES[d], ei.dtype)])
    zer = jnp.zeros((N_PAD["paper"], DP), jnp.float32)
    xd = {t: p["x_" + t] for t in NODE_TYPES}
    h1 = _layer(xd, srcp, dstp, zer, p, 1)
    h2 = _layer(h1, srcp, dstp, zer, p, 2)
    hcat = jnp.vstack([h2[t] for t in NODE_TYPES])
    return _final(hcat, lin_W, lin_b)


# speedup vs baseline: 47.1428x; 1.0368x over previous
import functools

import jax
import jax.numpy as jnp
import numpy as np
from jax import lax
from jax.experimental import pallas as pl
from jax.experimental.pallas import tpu as pltpu
from jax.experimental.pallas import tpu_sc as plsc

NODE_TYPES = ("paper", "author", "term")
N_NODES = {"paper": 4000, "author": 3000, "term": 3000}
N_PAD = {"paper": 4096, "author": 3072, "term": 3072}
EDGE_TYPES = (("paper", "author"), ("author", "paper"), ("paper", "term"), ("term", "paper"))
N_EDGES = 40000
E_PAD = 40960
H = 8
DIMS = {1: (2048, 128), 2: (128, 64)}
OUT_DIM = 16
DP = 128
NW = 32
CHUNK = 32
CHUNKS_PER_W = E_PAD // (NW * CHUNK)


def _prep_body(dh, n_src, *refs):
    wk, wq, wv, bk, bq, bv = refs[:6]
    rels = refs[6:6 + 2 * n_src]
    wf_ref, bf_ref = refs[6 + 2 * n_src:]

    def heads(mat, rel=None):
        cols = []
        for h in range(H):
            blk = mat[:, h * dh:(h + 1) * dh]
            if rel is not None:
                blk = jnp.dot(blk, rel[h], preferred_element_type=jnp.float32)
            cols.append(blk)
        return jnp.concatenate(cols, axis=1)

    wcols = [heads(wq[...])]
    bcols = [heads(bq[...])]
    for i in range(n_src):
        arel_s = rels[2 * i][...]
        mrel = rels[2 * i + 1][...]
        wcols += [heads(wk[...], arel_s), heads(wv[...], mrel)]
        bcols += [heads(bk[...], arel_s), heads(bv[...], mrel)]
    wf_ref[...] = jnp.concatenate(wcols, axis=1)
    bf_ref[...] = jnp.concatenate(bcols, axis=1)


def _prep(L, t, p, arel_s, mrel):
    din, dout = DIMS[L]
    dh = dout // H
    n_src = len(arel_s)
    K = dout * (1 + 2 * n_src)
    ins = [p["c%d_Wk_%s" % (L, t)], p["c%d_Wq_%s" % (L, t)], p["c%d_Wv_%s" % (L, t)],
           p["c%d_bk_%s" % (L, t)].reshape(1, dout), p["c%d_bq_%s" % (L, t)].reshape(1, dout),
           p["c%d_bv_%s" % (L, t)].reshape(1, dout)]
    for a, m in zip(arel_s, mrel):
        ins += [a, m]
    outs = [jax.ShapeDtypeStruct((din, K), jnp.float32), jax.ShapeDtypeStruct((1, K), jnp.float32)]
    return pl.pallas_call(
        functools.partial(_prep_body, dh, n_src),
        out_shape=tuple(outs),
    )(*ins)


def _proj_body(n_kv, dout, x_ref, wf_ref, bf_ref, q_ref, *kv_refs):
    y = jnp.dot(x_ref[...], wf_ref[...], preferred_element_type=jnp.float32) + bf_ref[...]
    qv = y[:, :dout]
    if dout < DP:
        qv = jnp.concatenate([qv, jnp.zeros((qv.shape[0], DP - dout), jnp.float32)], axis=1)
    q_ref[...] = qv
    for i in range(n_kv):
        kv_refs[i][...] = y[:, dout + 2 * dout * i:dout + 2 * dout * (i + 1)]


def _proj(x, wf, bf, n_kv, dout):
    n, din = x.shape
    K = wf.shape[1]
    R = 1000
    grid = n // R
    return pl.pallas_call(
        functools.partial(_proj_body, n_kv, dout),
        out_shape=tuple([jax.ShapeDtypeStruct((n, DP), jnp.float32)]
                        + [jax.ShapeDtypeStruct((n, 2 * dout), jnp.float32)] * n_kv),
        grid=(grid,),
        in_specs=[
            pl.BlockSpec((R, din), lambda i: (i, 0)),
            pl.BlockSpec((din, K), lambda i: (0, 0)),
            pl.BlockSpec((1, K), lambda i: (0, 0)),
        ],
        out_specs=tuple([pl.BlockSpec((R, DP), lambda i: (i, 0))]
                        + [pl.BlockSpec((R, 2 * dout), lambda i: (i, 0))] * n_kv),
    )(x, wf, bf)


def _edge_body(meta, D, *refs):
    kvs = refs[0:4]
    qs = refs[4:7]
    idx = refs[7:15]
    zer = refs[15]
    outs = refs[16:20]
    (dstg0, dstg1, dsts0, dsts1, srcall, dstall, kvb0, kvb1, qb0, qb1,
     exb0, exb1, wvb0, wvb1, sh_den, sh_num,
     skv0, skv1, sq0, sq1, sx0, sx1, sw0, sw1, sz0, sz1) = refs[20:]
    c = lax.axis_index("c")
    s = lax.axis_index("s")
    w = s * 2 + c
    lane = lax.broadcasted_iota(jnp.int32, (16,), 0)
    gdn = lax.GatherDimensionNumbers(offset_dims=(), collapsed_slice_dims=(0,), start_index_map=(0,))
    offsets = (8, 4, 2, 1) if D == 128 else (4, 2, 1)
    shuf_idx = [(lane ^ off).reshape(16, 1) for off in offsets]
    groups = D // 16

    def allsum(x):
        for idx in shuf_idx:
            x = x + lax.gather(x, idx, gdn, (1,),
                               mode=lax.GatherScatterMode.PROMISE_IN_BOUNDS)
        return x

    if D != DP:
        zv = jnp.zeros((16,), jnp.float32)

        def zrow(e, _):
            for buf in (exb0, exb1, wvb0, wvb1):
                for j in range(D // 16, DP // 16):
                    buf[e, pl.ds(j * 16, 16)] = zv
            return 0

        lax.fori_loop(0, CHUNK, zrow, 0)
    for ti, (qi, nd, ndp) in enumerate(meta):
        kv = kvs[ti]
        q = qs[qi]
        se, de = idx[2 * ti], idx[2 * ti + 1]
        out = outs[ti]
        rows = ndp // 16
        r0 = s * rows
        pltpu.async_copy(zer.at[pl.ds(r0, rows)], sh_den.at[pl.ds(r0, rows)], sz0)
        pltpu.async_copy(zer.at[pl.ds(r0, rows)], sh_num.at[pl.ds(r0, rows)], sz1)
        wspan = CHUNKS_PER_W * CHUNK
        pltpu.sync_copy(se.at[pl.ds(w * wspan, wspan)], srcall)
        pltpu.sync_copy(de.at[pl.ds(w * wspan, wspan)], dstall)
        pltpu.make_async_copy(zer.at[pl.ds(r0, rows)], sh_den.at[pl.ds(r0, rows)], sz0).wait()
        pltpu.make_async_copy(zer.at[pl.ds(r0, rows)], sh_num.at[pl.ds(r0, rows)], sz1).wait()
        plsc.subcore_barrier()

        def fire(g, dg, kb, qbf, sk, sq):
            off = g * CHUNK
            for i in range(CHUNK // 16):
                dg[pl.ds(i * 16, 16)] = jnp.minimum(dstall[pl.ds(off + i * 16, 16)], nd - 1)
            pltpu.async_copy(kv.at[srcall.at[pl.ds(off, CHUNK)]], kb, sk)
            pltpu.async_copy(q.at[dg], qbf, sq)

        def drain_compute_scatter(p, g, dg, kb, qbf, sk, sq, xb, wb, ds_, sx, sw):
            off = g * CHUNK
            pltpu.make_async_copy(kv.at[srcall.at[pl.ds(off, CHUNK)]], kb, sk).wait()
            pltpu.make_async_copy(q.at[dg], qbf, sq).wait()

            @pl.when(p > 0)
            def _():
                pltpu.make_async_copy(xb, sh_den.at[ds_], sx).wait()
                pltpu.make_async_copy(wb, sh_num.at[ds_], sw).wait()

            for i in range(CHUNK // 16):
                ds_[pl.ds(i * 16, 16)] = dstall[pl.ds(off + i * 16, 16)]

            @plsc.parallel_loop(0, CHUNK, 1, unroll=2)
            def edge(e):
                for h in range(groups):
                    kvv = kb[e, pl.ds(h * 16, 16)]
                    qv = qbf[e, pl.ds(h * 16, 16)]
                    exv = jnp.exp(allsum(qv * kvv))
                    xb[e, pl.ds(h * 16, 16)] = exv
                    wb[e, pl.ds(h * 16, 16)] = exv * kb[e, pl.ds(D + h * 16, 16)]

            pltpu.async_copy(xb, sh_den.at[ds_], sx, add=True)
            pltpu.async_copy(wb, sh_num.at[ds_], sw, add=True)

        bufs0 = (dstg0, kvb0, qb0, skv0, sq0)
        bufs1 = (dstg1, kvb1, qb1, skv1, sq1)
        sc0 = (exb0, wvb0, dsts0, sx0, sw0)
        sc1 = (exb1, wvb1, dsts1, sx1, sw1)
        pairs = CHUNKS_PER_W // 2
        fire(0, *bufs0)

        def pair(p, _):
            fire(2 * p + 1, *bufs1)
            drain_compute_scatter(p, 2 * p, *bufs0, *sc0)

            @pl.when(p < pairs - 1)
            def _():
                fire(2 * p + 2, *bufs0)

            drain_compute_scatter(p, 2 * p + 1, *bufs1, *sc1)
            return 0

        lax.fori_loop(0, pairs, pair, 0)
        pltpu.make_async_copy(exb0, sh_den.at[dsts0], sx0).wait()
        pltpu.make_async_copy(wvb0, sh_num.at[dsts0], sw0).wait()
        pltpu.make_async_copy(exb1, sh_den.at[dsts1], sx1).wait()
        pltpu.make_async_copy(wvb1, sh_num.at[dsts1], sw1).wait()
        plsc.subcore_barrier()
        pltpu.sync_copy(sh_den.at[pl.ds(r0, rows)], out.at[c, 0, pl.ds(r0, rows)])
        pltpu.sync_copy(sh_num.at[pl.ds(r0, rows)], out.at[c, 1, pl.ds(r0, rows)])
        plsc.subcore_barrier()


def _edge_stage(kv, q, srcp, dstp, zer, D):
    meta = []
    ins = []
    for (s, d) in EDGE_TYPES:
        en = s + "__" + d
        ins.append(kv[en])
        meta.append((NODE_TYPES.index(d), N_NODES[d], N_PAD[d]))
    for t in NODE_TYPES:
        ins.append(q[t])
    for (s, d) in EDGE_TYPES:
        en = s + "__" + d
        ins += [srcp[en], dstp[en]]
    ins.append(zer)
    out_type = tuple(jax.ShapeDtypeStruct((2, 2, N_PAD[d], DP), jnp.float32) for (s, d) in EDGE_TYPES)
    mesh = plsc.VectorSubcoreMesh(core_axis_name="c", subcore_axis_name="s")
    f = pl.kernel(
        functools.partial(_edge_body, meta, D),
        out_type=out_type,
        mesh=mesh,
        scratch_types=[pltpu.VMEM((CHUNK,), jnp.int32)] * 4
        + [pltpu.VMEM((CHUNKS_PER_W * CHUNK,), jnp.int32)] * 2
        + [pltpu.VMEM((CHUNK, 2 * D), jnp.float32)] * 2
        + [pltpu.VMEM((CHUNK, DP), jnp.float32)] * 6
        + [pltpu.VMEM_SHARED((N_PAD["paper"], DP), jnp.float32)] * 2
        + [pltpu.SemaphoreType.DMA] * 10,
    )
    return f(*ins)


def _post_body(n_en, dout, slab0, *refs):
    slabs = (slab0,) + refs[:n_en - 1]
    wa_ref, ba_ref, out_ref = refs[n_en - 1:]
    agg = None
    for sl_ref in slabs:
        sl = sl_ref[...]
        den = sl[0, 0, :, :dout] + sl[1, 0, :, :dout]
        num = sl[0, 1, :, :dout] + sl[1, 1, :, :dout]
        part = num / (den + 1e-16)
        agg = part if agg is None else agg + part
    g = 0.5 * agg * (1.0 + jnp.tanh(0.7978845608028654 * (agg + 0.044715 * agg * agg * agg)))
    out_ref[...] = jnp.dot(g, wa_ref[...], preferred_element_type=jnp.float32) + ba_ref[...]


def _post(slabs, wa, ba, n):
    n_en = len(slabs)
    dout = wa.shape[1]
    R = 1000
    grid = n // R
    return pl.pallas_call(
        functools.partial(_post_body, n_en, dout),
        out_shape=jax.ShapeDtypeStruct((n, dout), jnp.float32),
        grid=(grid,),
        in_specs=[pl.BlockSpec((2, 2, R, DP), lambda i: (0, 0, i, 0))] * n_en
        + [pl.BlockSpec((dout, dout), lambda i: (0, 0)),
           pl.BlockSpec((1, dout), lambda i: (0, 0))],
        out_specs=pl.BlockSpec((R, dout), lambda i: (i, 0)),
    )(*slabs, wa, ba.reshape(1, dout))


def _final_body(h_ref, w_ref, b_ref, o_ref):
    logits = jnp.dot(h_ref[...], w_ref[...], preferred_element_type=jnp.float32) + b_ref[...]
    m = jnp.max(logits, axis=1, keepdims=True)
    e = jnp.exp(logits - m)
    o_ref[...] = e / jnp.sum(e, axis=1, keepdims=True)


def _final(h, lin_W, lin_b):
    n = h.shape[0]
    return pl.pallas_call(
        _final_body,
        out_shape=jax.ShapeDtypeStruct((n, OUT_DIM), jnp.float32),
        grid=(10,),
        in_specs=[
            pl.BlockSpec((n // 10, 64), lambda i: (i, 0)),
            pl.BlockSpec((64, OUT_DIM), lambda i: (0, 0)),
            pl.BlockSpec((1, OUT_DIM), lambda i: (0, 0)),
        ],
        out_specs=pl.BlockSpec((n // 10, OUT_DIM), lambda i: (i, 0)),
    )(h, lin_W, lin_b.reshape(1, OUT_DIM))


def _layer(xd, srcp, dstp, zer, p, L):
    din, dout = DIMS[L]
    dh = dout // H
    scale = 1.0 / np.sqrt(dh)
    src_of = {t: [en for en in EDGE_TYPES if en[0] == t] for t in NODE_TYPES}
    dst_of = {t: [en for en in EDGE_TYPES if en[1] == t] for t in NODE_TYPES}
    q = {}
    kv = {}
    for t in NODE_TYPES:
        arel_s = []
        mrel = []
        for (s, d) in src_of[t]:
            en = s + "__" + d
            arel_s.append(p["c%d_arel_%s" % (L, en)]
                          * (p["c%d_prel_%s" % (L, en)] * scale)[:, None, None])
            mrel.append(p["c%d_mrel_%s" % (L, en)])
        wf, bf = _prep(L, t, p, arel_s, mrel)
        outs = _proj(xd[t], wf, bf, len(src_of[t]), dout)
        q[t] = outs[0]
        for i, (s, d) in enumerate(src_of[t]):
            kv[s + "__" + d] = outs[1 + i]
    slabs = _edge_stage(kv, q, srcp, dstp, zer, dout)
    slab_of = {s + "__" + d: slabs[i] for i, (s, d) in enumerate(EDGE_TYPES)}
    out = {}
    for t in NODE_TYPES:
        sl = [slab_of[s + "__" + d] for (s, d) in dst_of[t]]
        out[t] = _post(sl, p["c%d_Wa_%s" % (L, t)], p["c%d_ba_%s" % (L, t)], N_NODES[t])
    return out


def kernel(x_paper, x_author, x_term, edge_index_paper__author, edge_index_author__paper, edge_index_paper__term, edge_index_term__paper, c1_Wk_paper, c1_Wq_paper, c1_Wv_paper, c1_bk_paper, c1_bq_paper, c1_bv_paper, c1_Wa_paper, c1_ba_paper, c1_Wk_author, c1_Wq_author, c1_Wv_author, c1_bk_author, c1_bq_author, c1_bv_author, c1_Wa_author, c1_ba_author, c1_Wk_term, c1_Wq_term, c1_Wv_term, c1_bk_term, c1_bq_term, c1_bv_term, c1_Wa_term, c1_ba_term, c1_arel_paper__author, c1_mrel_paper__author, c1_prel_paper__author, c1_arel_author__paper, c1_mrel_author__paper, c1_prel_author__paper, c1_arel_paper__term, c1_mrel_paper__term, c1_prel_paper__term, c1_arel_term__paper, c1_mrel_term__paper, c1_prel_term__paper, c2_Wk_paper, c2_Wq_paper, c2_Wv_paper, c2_bk_paper, c2_bq_paper, c2_bv_paper, c2_Wa_paper, c2_ba_paper, c2_Wk_author, c2_Wq_author, c2_Wv_author, c2_bk_author, c2_bq_author, c2_bv_author, c2_Wa_author, c2_ba_author, c2_Wk_term, c2_Wq_term, c2_Wv_term, c2_bk_term, c2_bq_term, c2_bv_term, c2_Wa_term, c2_ba_term, c2_arel_paper__author, c2_mrel_paper__author, c2_prel_paper__author, c2_arel_author__paper, c2_mrel_author__paper, c2_prel_author__paper, c2_arel_paper__term, c2_mrel_paper__term, c2_prel_paper__term, c2_arel_term__paper, c2_mrel_term__paper, c2_prel_term__paper, lin_W, lin_b):
    kwargs = dict(locals())
    p = {kk: vv for kk, vv in kwargs.items() if not kk.startswith("edge_index_")}
    srcp = {}
    dstp = {}
    for (s, d) in EDGE_TYPES:
        en = s + "__" + d
        ei = kwargs["edge_index_" + en]
        pad = E_PAD - N_EDGES
        srcp[en] = jnp.concatenate([ei[0], jnp.zeros((pad,), ei.dtype)])
        dstp[en] = jnp.concatenate([ei[1], jnp.full((pad,), N_NODES[d], ei.dtype)])
    zer = jnp.zeros((N_PAD["paper"], DP), jnp.float32)
    xd = {t: p["x_" + t] for t in NODE_TYPES}
    h1 = _layer(xd, srcp, dstp, zer, p, 1)
    h2 = _layer(h1, srcp, dstp, zer, p, 2)
    hcat = jnp.vstack([h2[t] for t in NODE_TYPES])
    return _final(hcat, lin_W, lin_b)


# speedup vs baseline: 48.4021x; 1.0267x over previous
import functools

import jax
import jax.numpy as jnp
import numpy as np
from jax import lax
from jax.experimental import pallas as pl
from jax.experimental.pallas import tpu as pltpu
from jax.experimental.pallas import tpu_sc as plsc

NODE_TYPES = ("paper", "author", "term")
N_NODES = {"paper": 4000, "author": 3000, "term": 3000}
N_PAD = {"paper": 4096, "author": 3072, "term": 3072}
EDGE_TYPES = (("paper", "author"), ("author", "paper"), ("paper", "term"), ("term", "paper"))
N_EDGES = 40000
E_PAD = 40960
H = 8
DIMS = {1: (2048, 128), 2: (128, 64)}
OUT_DIM = 16
DP = 128
NW = 32
CHUNK = 32
CHUNKS_PER_W = E_PAD // (NW * CHUNK)


def _blockdiag(rel):
    Hh, dh, _ = rel.shape
    eye = jnp.eye(Hh, dtype=rel.dtype)[:, None, :, None]
    return (eye * rel[:, :, None, :]).reshape(Hh * dh, Hh * dh)


def _prep_body(dout, wq, wk, wv, bq, bk, bv, g_ref, wf_ref, bf_ref):
    g = g_ref[...]
    g0, g1, g2 = g[:dout], g[dout:2 * dout], g[2 * dout:]
    dot = functools.partial(jnp.dot, preferred_element_type=jnp.float32)
    wf_ref[...] = dot(wq[...], g0) + dot(wk[...], g1) + dot(wv[...], g2)
    bf_ref[...] = dot(bq[...], g0) + dot(bk[...], g1) + dot(bv[...], g2)


def _prep(L, t, p, arel_s, mrel):
    din, dout = DIMS[L]
    n_src = len(arel_s)
    K = dout * (1 + 2 * n_src)
    g = jnp.zeros((3 * dout, K), jnp.float32)
    g = g.at[:dout, :dout].set(jnp.eye(dout, dtype=jnp.float32))
    for i in range(n_src):
        c0 = dout * (1 + 2 * i)
        g = g.at[dout:2 * dout, c0:c0 + dout].set(_blockdiag(arel_s[i]))
        g = g.at[2 * dout:, c0 + dout:c0 + 2 * dout].set(_blockdiag(mrel[i]))
    ins = [p["c%d_Wq_%s" % (L, t)], p["c%d_Wk_%s" % (L, t)], p["c%d_Wv_%s" % (L, t)],
           p["c%d_bq_%s" % (L, t)].reshape(1, dout), p["c%d_bk_%s" % (L, t)].reshape(1, dout),
           p["c%d_bv_%s" % (L, t)].reshape(1, dout), g]
    outs = [jax.ShapeDtypeStruct((din, K), jnp.float32), jax.ShapeDtypeStruct((1, K), jnp.float32)]
    return pl.pallas_call(
        functools.partial(_prep_body, dout),
        out_shape=tuple(outs),
    )(*ins)


def _proj_body(n_kv, dout, x_ref, wf_ref, bf_ref, q_ref, *kv_refs):
    y = jnp.dot(x_ref[...], wf_ref[...], preferred_element_type=jnp.float32) + bf_ref[...]
    qv = y[:, :dout]
    if dout < DP:
        qv = jnp.concatenate([qv, jnp.zeros((qv.shape[0], DP - dout), jnp.float32)], axis=1)
    q_ref[...] = qv
    for i in range(n_kv):
        kv_refs[i][...] = y[:, dout + 2 * dout * i:dout + 2 * dout * (i + 1)]


def _proj(x, wf, bf, n_kv, dout):
    n, din = x.shape
    K = wf.shape[1]
    R = 1000
    grid = n // R
    return pl.pallas_call(
        functools.partial(_proj_body, n_kv, dout),
        out_shape=tuple([jax.ShapeDtypeStruct((n, DP), jnp.float32)]
                        + [jax.ShapeDtypeStruct((n, 2 * dout), jnp.float32)] * n_kv),
        grid=(grid,),
        in_specs=[
            pl.BlockSpec((R, din), lambda i: (i, 0)),
            pl.BlockSpec((din, K), lambda i: (0, 0)),
            pl.BlockSpec((1, K), lambda i: (0, 0)),
        ],
        out_specs=tuple([pl.BlockSpec((R, DP), lambda i: (i, 0))]
                        + [pl.BlockSpec((R, 2 * dout), lambda i: (i, 0))] * n_kv),
    )(x, wf, bf)


def _edge_body(meta, D, *refs):
    kvs = refs[0:4]
    qs = refs[4:7]
    idx = refs[7:15]
    zer = refs[15]
    outs = refs[16:20]
    (dstg0, dstg1, dsts0, dsts1, srcall, dstall, kvb0, kvb1, qb0, qb1,
     exb0, exb1, wvb0, wvb1, sh_den, sh_num,
     skv0, skv1, sq0, sq1, sx0, sx1, sw0, sw1, sz0, sz1) = refs[20:]
    c = lax.axis_index("c")
    s = lax.axis_index("s")
    w = s * 2 + c
    lane = lax.broadcasted_iota(jnp.int32, (16,), 0)
    gdn = lax.GatherDimensionNumbers(offset_dims=(), collapsed_slice_dims=(0,), start_index_map=(0,))
    offsets = (8, 4, 2, 1) if D == 128 else (4, 2, 1)
    shuf_idx = [(lane ^ off).reshape(16, 1) for off in offsets]
    groups = D // 16

    def allsum(x):
        for idx in shuf_idx:
            x = x + lax.gather(x, idx, gdn, (1,),
                               mode=lax.GatherScatterMode.PROMISE_IN_BOUNDS)
        return x

    if D != DP:
        zv = jnp.zeros((16,), jnp.float32)

        def zrow(e, _):
            for buf in (exb0, exb1, wvb0, wvb1):
                for j in range(D // 16, DP // 16):
                    buf[e, pl.ds(j * 16, 16)] = zv
            return 0

        lax.fori_loop(0, CHUNK, zrow, 0)
    for ti, (qi, nd, ndp) in enumerate(meta):
        kv = kvs[ti]
        q = qs[qi]
        se, de = idx[2 * ti], idx[2 * ti + 1]
        out = outs[ti]
        rows = ndp // 16
        r0 = s * rows
        pltpu.async_copy(zer.at[pl.ds(r0, rows)], sh_den.at[pl.ds(r0, rows)], sz0)
        pltpu.async_copy(zer.at[pl.ds(r0, rows)], sh_num.at[pl.ds(r0, rows)], sz1)
        wspan = CHUNKS_PER_W * CHUNK
        pltpu.sync_copy(se.at[pl.ds(w * wspan, wspan)], srcall)
        pltpu.sync_copy(de.at[pl.ds(w * wspan, wspan)], dstall)
        pltpu.make_async_copy(zer.at[pl.ds(r0, rows)], sh_den.at[pl.ds(r0, rows)], sz0).wait()
        pltpu.make_async_copy(zer.at[pl.ds(r0, rows)], sh_num.at[pl.ds(r0, rows)], sz1).wait()
        plsc.subcore_barrier()

        def fire(g, dg, kb, qbf, sk, sq):
            off = g * CHUNK
            for i in range(CHUNK // 16):
                dg[pl.ds(i * 16, 16)] = jnp.minimum(dstall[pl.ds(off + i * 16, 16)], nd - 1)
            pltpu.async_copy(kv.at[srcall.at[pl.ds(off, CHUNK)]], kb, sk)
            pltpu.async_copy(q.at[dg], qbf, sq)

        def drain_compute_scatter(p, g, dg, kb, qbf, sk, sq, xb, wb, ds_, sx, sw):
            off = g * CHUNK
            pltpu.make_async_copy(kv.at[srcall.at[pl.ds(off, CHUNK)]], kb, sk).wait()
            pltpu.make_async_copy(q.at[dg], qbf, sq).wait()

            @pl.when(p > 0)
            def _():
                pltpu.make_async_copy(xb, sh_den.at[ds_], sx).wait()
                pltpu.make_async_copy(wb, sh_num.at[ds_], sw).wait()

            for i in range(CHUNK // 16):
                ds_[pl.ds(i * 16, 16)] = dstall[pl.ds(off + i * 16, 16)]

            @plsc.parallel_loop(0, CHUNK, 1, unroll=2)
            def edge(e):
                for h in range(groups):
                    kvv = kb[e, pl.ds(h * 16, 16)]
                    qv = qbf[e, pl.ds(h * 16, 16)]
                    exv = jnp.exp(allsum(qv * kvv))
                    xb[e, pl.ds(h * 16, 16)] = exv
                    wb[e, pl.ds(h * 16, 16)] = exv * kb[e, pl.ds(D + h * 16, 16)]

            pltpu.async_copy(xb, sh_den.at[ds_], sx, add=True)
            pltpu.async_copy(wb, sh_num.at[ds_], sw, add=True)

        bufs0 = (dstg0, kvb0, qb0, skv0, sq0)
        bufs1 = (dstg1, kvb1, qb1, skv1, sq1)
        sc0 = (exb0, wvb0, dsts0, sx0, sw0)
        sc1 = (exb1, wvb1, dsts1, sx1, sw1)
        pairs = CHUNKS_PER_W // 2
        fire(0, *bufs0)

        def pair(p, _):
            fire(2 * p + 1, *bufs1)
            drain_compute_scatter(p, 2 * p, *bufs0, *sc0)

            @pl.when(p < pairs - 1)
            def _():
                fire(2 * p + 2, *bufs0)

            drain_compute_scatter(p, 2 * p + 1, *bufs1, *sc1)
            return 0

        lax.fori_loop(0, pairs, pair, 0)
        pltpu.make_async_copy(exb0, sh_den.at[dsts0], sx0).wait()
        pltpu.make_async_copy(wvb0, sh_num.at[dsts0], sw0).wait()
        pltpu.make_async_copy(exb1, sh_den.at[dsts1], sx1).wait()
        pltpu.make_async_copy(wvb1, sh_num.at[dsts1], sw1).wait()
        plsc.subcore_barrier()
        pltpu.sync_copy(sh_den.at[pl.ds(r0, rows)], out.at[c, 0, pl.ds(r0, rows)])
        pltpu.sync_copy(sh_num.at[pl.ds(r0, rows)], out.at[c, 1, pl.ds(r0, rows)])
        plsc.subcore_barrier()


def _edge_stage(kv, q, srcp, dstp, zer, D):
    meta = []
    ins = []
    for (s, d) in EDGE_TYPES:
        en = s + "__" + d
        ins.append(kv[en])
        meta.append((NODE_TYPES.index(d), N_NODES[d], N_PAD[d]))
    for t in NODE_TYPES:
        ins.append(q[t])
    for (s, d) in EDGE_TYPES:
        en = s + "__" + d
        ins += [srcp[en], dstp[en]]
    ins.append(zer)
    out_type = tuple(jax.ShapeDtypeStruct((2, 2, N_PAD[d], DP), jnp.float32) for (s, d) in EDGE_TYPES)
    mesh = plsc.VectorSubcoreMesh(core_axis_name="c", subcore_axis_name="s")
    f = pl.kernel(
        functools.partial(_edge_body, meta, D),
        out_type=out_type,
        mesh=mesh,
        scratch_types=[pltpu.VMEM((CHUNK,), jnp.int32)] * 4
        + [pltpu.VMEM((CHUNKS_PER_W * CHUNK,), jnp.int32)] * 2
        + [pltpu.VMEM((CHUNK, 2 * D), jnp.float32)] * 2
        + [pltpu.VMEM((CHUNK, DP), jnp.float32)] * 6
        + [pltpu.VMEM_SHARED((N_PAD["paper"], DP), jnp.float32)] * 2
        + [pltpu.SemaphoreType.DMA] * 10,
    )
    return f(*ins)


def _post_body(n_en, dout, slab0, *refs):
    slabs = (slab0,) + refs[:n_en - 1]
    wa_ref, ba_ref, out_ref = refs[n_en - 1:]
    agg = None
    for sl_ref in slabs:
        sl = sl_ref[...]
        den = sl[0, 0, :, :dout] + sl[1, 0, :, :dout]
        num = sl[0, 1, :, :dout] + sl[1, 1, :, :dout]
        part = num / (den + 1e-16)
        agg = part if agg is None else agg + part
    g = 0.5 * agg * (1.0 + jnp.tanh(0.7978845608028654 * (agg + 0.044715 * agg * agg * agg)))
    out_ref[...] = jnp.dot(g, wa_ref[...], preferred_element_type=jnp.float32) + ba_ref[...]


def _post(slabs, wa, ba, n):
    n_en = len(slabs)
    dout = wa.shape[1]
    R = 1000
    grid = n // R
    return pl.pallas_call(
        functools.partial(_post_body, n_en, dout),
        out_shape=jax.ShapeDtypeStruct((n, dout), jnp.float32),
        grid=(grid,),
        in_specs=[pl.BlockSpec((2, 2, R, DP), lambda i: (0, 0, i, 0))] * n_en
        + [pl.BlockSpec((dout, dout), lambda i: (0, 0)),
           pl.BlockSpec((1, dout), lambda i: (0, 0))],
        out_specs=pl.BlockSpec((R, dout), lambda i: (i, 0)),
    )(*slabs, wa, ba.reshape(1, dout))


def _final_body(h_ref, w_ref, b_ref, o_ref):
    logits = jnp.dot(h_ref[...], w_ref[...], preferred_element_type=jnp.float32) + b_ref[...]
    m = jnp.max(logits, axis=1, keepdims=True)
    e = jnp.exp(logits - m)
    o_ref[...] = e / jnp.sum(e, axis=1, keepdims=True)


def _final(h, lin_W, lin_b):
    n = h.shape[0]
    return pl.pallas_call(
        _final_body,
        out_shape=jax.ShapeDtypeStruct((n, OUT_DIM), jnp.float32),
        grid=(10,),
        in_specs=[
            pl.BlockSpec((n // 10, 64), lambda i: (i, 0)),
            pl.BlockSpec((64, OUT_DIM), lambda i: (0, 0)),
            pl.BlockSpec((1, OUT_DIM), lambda i: (0, 0)),
        ],
        out_specs=pl.BlockSpec((n // 10, OUT_DIM), lambda i: (i, 0)),
    )(h, lin_W, lin_b.reshape(1, OUT_DIM))


def _layer(xd, srcp, dstp, zer, p, L):
    din, dout = DIMS[L]
    dh = dout // H
    scale = 1.0 / np.sqrt(dh)
    src_of = {t: [en for en in EDGE_TYPES if en[0] == t] for t in NODE_TYPES}
    dst_of = {t: [en for en in EDGE_TYPES if en[1] == t] for t in NODE_TYPES}
    q = {}
    kv = {}
    for t in NODE_TYPES:
        arel_s = []
        mrel = []
        for (s, d) in src_of[t]:
            en = s + "__" + d
            arel_s.append(p["c%d_arel_%s" % (L, en)]
                          * (p["c%d_prel_%s" % (L, en)] * scale)[:, None, None])
            mrel.append(p["c%d_mrel_%s" % (L, en)])
        wf, bf = _prep(L, t, p, arel_s, mrel)
        outs = _proj(xd[t], wf, bf, len(src_of[t]), dout)
        q[t] = outs[0]
        for i, (s, d) in enumerate(src_of[t]):
            kv[s + "__" + d] = outs[1 + i]
    slabs = _edge_stage(kv, q, srcp, dstp, zer, dout)
    slab_of = {s + "__" + d: slabs[i] for i, (s, d) in enumerate(EDGE_TYPES)}
    out = {}
    for t in NODE_TYPES:
        sl = [slab_of[s + "__" + d] for (s, d) in dst_of[t]]
        out[t] = _post(sl, p["c%d_Wa_%s" % (L, t)], p["c%d_ba_%s" % (L, t)], N_NODES[t])
    return out


def kernel(x_paper, x_author, x_term, edge_index_paper__author, edge_index_author__paper, edge_index_paper__term, edge_index_term__paper, c1_Wk_paper, c1_Wq_paper, c1_Wv_paper, c1_bk_paper, c1_bq_paper, c1_bv_paper, c1_Wa_paper, c1_ba_paper, c1_Wk_author, c1_Wq_author, c1_Wv_author, c1_bk_author, c1_bq_author, c1_bv_author, c1_Wa_author, c1_ba_author, c1_Wk_term, c1_Wq_term, c1_Wv_term, c1_bk_term, c1_bq_term, c1_bv_term, c1_Wa_term, c1_ba_term, c1_arel_paper__author, c1_mrel_paper__author, c1_prel_paper__author, c1_arel_author__paper, c1_mrel_author__paper, c1_prel_author__paper, c1_arel_paper__term, c1_mrel_paper__term, c1_prel_paper__term, c1_arel_term__paper, c1_mrel_term__paper, c1_prel_term__paper, c2_Wk_paper, c2_Wq_paper, c2_Wv_paper, c2_bk_paper, c2_bq_paper, c2_bv_paper, c2_Wa_paper, c2_ba_paper, c2_Wk_author, c2_Wq_author, c2_Wv_author, c2_bk_author, c2_bq_author, c2_bv_author, c2_Wa_author, c2_ba_author, c2_Wk_term, c2_Wq_term, c2_Wv_term, c2_bk_term, c2_bq_term, c2_bv_term, c2_Wa_term, c2_ba_term, c2_arel_paper__author, c2_mrel_paper__author, c2_prel_paper__author, c2_arel_author__paper, c2_mrel_author__paper, c2_prel_author__paper, c2_arel_paper__term, c2_mrel_paper__term, c2_prel_paper__term, c2_arel_term__paper, c2_mrel_term__paper, c2_prel_term__paper, lin_W, lin_b):
    kwargs = dict(locals())
    p = {kk: vv for kk, vv in kwargs.items() if not kk.startswith("edge_index_")}
    srcp = {}
    dstp = {}
    for (s, d) in EDGE_TYPES:
        en = s + "__" + d
        ei = kwargs["edge_index_" + en]
        pad = E_PAD - N_EDGES
        srcp[en] = jnp.concatenate([ei[0], jnp.zeros((pad,), ei.dtype)])
        dstp[en] = jnp.concatenate([ei[1], jnp.full((pad,), N_NODES[d], ei.dtype)])
    zer = jnp.zeros((N_PAD["paper"], DP), jnp.float32)
    xd = {t: p["x_" + t] for t in NODE_TYPES}
    h1 = _layer(xd, srcp, dstp, zer, p, 1)
    h2 = _layer(h1, srcp, dstp, zer, p, 2)
    hcat = jnp.vstack([h2[t] for t in NODE_TYPES])
    return _final(hcat, lin_W, lin_b)


# speedup vs baseline: 48.4379x; 1.0007x over previous
import functools

import jax
import jax.numpy as jnp
import numpy as np
from jax import lax
from jax.experimental import pallas as pl
from jax.experimental.pallas import tpu as pltpu
from jax.experimental.pallas import tpu_sc as plsc

NODE_TYPES = ("paper", "author", "term")
N_NODES = {"paper": 4000, "author": 3000, "term": 3000}
N_PAD = {"paper": 4096, "author": 3072, "term": 3072}
EDGE_TYPES = (("paper", "author"), ("author", "paper"), ("paper", "term"), ("term", "paper"))
N_EDGES = 40000
E_PAD = 40960
H = 8
DIMS = {1: (2048, 128), 2: (128, 64)}
OUT_DIM = 16
DP = 128
NW = 32
CHUNK = 32
CHUNKS_PER_W = E_PAD // (NW * CHUNK)


def _blockdiag(rel):
    Hh, dh, _ = rel.shape
    eye = jnp.eye(Hh, dtype=rel.dtype)[:, None, :, None]
    return (eye * rel[:, :, None, :]).reshape(Hh * dh, Hh * dh)


def _prep_body(dout, wq, wk, wv, bq, bk, bv, g_ref, wf_ref, bf_ref):
    g = g_ref[...]
    g0, g1, g2 = g[:dout], g[dout:2 * dout], g[2 * dout:]
    dot = functools.partial(jnp.dot, preferred_element_type=jnp.float32)
    wf_ref[...] = dot(wq[...], g0) + dot(wk[...], g1) + dot(wv[...], g2)
    bf_ref[...] = dot(bq[...], g0) + dot(bk[...], g1) + dot(bv[...], g2)


def _prep(L, t, p, arel_s, mrel):
    din, dout = DIMS[L]
    n_src = len(arel_s)
    K = dout * (1 + 2 * n_src)
    g = jnp.zeros((3 * dout, K), jnp.float32)
    g = g.at[:dout, :dout].set(jnp.eye(dout, dtype=jnp.float32))
    for i in range(n_src):
        c0 = dout * (1 + 2 * i)
        g = g.at[dout:2 * dout, c0:c0 + dout].set(_blockdiag(arel_s[i]))
        g = g.at[2 * dout:, c0 + dout:c0 + 2 * dout].set(_blockdiag(mrel[i]))
    ins = [p["c%d_Wq_%s" % (L, t)], p["c%d_Wk_%s" % (L, t)], p["c%d_Wv_%s" % (L, t)],
           p["c%d_bq_%s" % (L, t)].reshape(1, dout), p["c%d_bk_%s" % (L, t)].reshape(1, dout),
           p["c%d_bv_%s" % (L, t)].reshape(1, dout), g]
    outs = [jax.ShapeDtypeStruct((din, K), jnp.float32), jax.ShapeDtypeStruct((1, K), jnp.float32)]
    return pl.pallas_call(
        functools.partial(_prep_body, dout),
        out_shape=tuple(outs),
    )(*ins)


def _proj_body(n_kv, dout, x_ref, wf_ref, bf_ref, q_ref, *kv_refs):
    y = jnp.dot(x_ref[...], wf_ref[...], preferred_element_type=jnp.float32) + bf_ref[...]
    qv = y[:, :dout]
    if dout < DP:
        qv = jnp.concatenate([qv, jnp.zeros((qv.shape[0], DP - dout), jnp.float32)], axis=1)
    q_ref[...] = qv
    for i in range(n_kv):
        kv_refs[i][...] = y[:, dout + 2 * dout * i:dout + 2 * dout * (i + 1)]


def _proj(x, wf, bf, n_kv, dout):
    n, din = x.shape
    K = wf.shape[1]
    R = 1000
    grid = n // R
    return pl.pallas_call(
        functools.partial(_proj_body, n_kv, dout),
        out_shape=tuple([jax.ShapeDtypeStruct((n, DP), jnp.float32)]
                        + [jax.ShapeDtypeStruct((n, 2 * dout), jnp.float32)] * n_kv),
        grid=(grid,),
        in_specs=[
            pl.BlockSpec((R, din), lambda i: (i, 0)),
            pl.BlockSpec((din, K), lambda i: (0, 0)),
            pl.BlockSpec((1, K), lambda i: (0, 0)),
        ],
        out_specs=tuple([pl.BlockSpec((R, DP), lambda i: (i, 0))]
                        + [pl.BlockSpec((R, 2 * dout), lambda i: (i, 0))] * n_kv),
    )(x, wf, bf)


def _edge_body(meta, D, *refs):
    kvs = refs[0:4]
    qs = refs[4:7]
    idx = refs[7:15]
    zer = refs[15]
    outs = refs[16:20]
    (dstg0, dstg1, dsts0, dsts1, srcall, dstall, kvb0, kvb1, qb0, qb1,
     exb0, exb1, wvb0, wvb1, sh_den, sh_num,
     skv0, skv1, sq0, sq1, sx0, sx1, sw0, sw1, sz0, sz1) = refs[20:]
    c = lax.axis_index("c")
    s = lax.axis_index("s")
    w = s * 2 + c
    lane = lax.broadcasted_iota(jnp.int32, (16,), 0)
    gdn = lax.GatherDimensionNumbers(offset_dims=(), collapsed_slice_dims=(0,), start_index_map=(0,))
    offsets = (8, 4, 2, 1) if D == 128 else (4, 2, 1)
    shuf_idx = [(lane ^ off).reshape(16, 1) for off in offsets]
    groups = D // 16

    def allsum(x):
        for idx in shuf_idx:
            x = x + lax.gather(x, idx, gdn, (1,),
                               mode=lax.GatherScatterMode.PROMISE_IN_BOUNDS)
        return x

    if D != DP:
        zv = jnp.zeros((16,), jnp.float32)

        def zrow(e, _):
            for buf in (exb0, exb1, wvb0, wvb1):
                for j in range(D // 16, DP // 16):
                    buf[e, pl.ds(j * 16, 16)] = zv
            return 0

        lax.fori_loop(0, CHUNK, zrow, 0)
    for ti, (qi, nd, ndp) in enumerate(meta):
        kv = kvs[ti]
        q = qs[qi]
        se, de = idx[2 * ti], idx[2 * ti + 1]
        out = outs[ti]
        rows = ndp // 16
        r0 = s * rows
        pltpu.async_copy(zer.at[pl.ds(r0, rows)], sh_den.at[pl.ds(r0, rows)], sz0)
        pltpu.async_copy(zer.at[pl.ds(r0, rows)], sh_num.at[pl.ds(r0, rows)], sz1)
        wspan = CHUNKS_PER_W * CHUNK
        pltpu.sync_copy(se.at[pl.ds(w * wspan, wspan)], srcall)
        pltpu.sync_copy(de.at[pl.ds(w * wspan, wspan)], dstall)
        pltpu.make_async_copy(zer.at[pl.ds(r0, rows)], sh_den.at[pl.ds(r0, rows)], sz0).wait()
        pltpu.make_async_copy(zer.at[pl.ds(r0, rows)], sh_num.at[pl.ds(r0, rows)], sz1).wait()
        plsc.subcore_barrier()

        def fire(g, dg, kb, qbf, sk, sq):
            off = g * CHUNK
            for i in range(CHUNK // 16):
                dg[pl.ds(i * 16, 16)] = jnp.minimum(dstall[pl.ds(off + i * 16, 16)], nd - 1)
            pltpu.async_copy(kv.at[srcall.at[pl.ds(off, CHUNK)]], kb, sk)
            pltpu.async_copy(q.at[dg], qbf, sq)

        def drain_compute_scatter(p, g, dg, kb, qbf, sk, sq, xb, wb, ds_, sx, sw):
            off = g * CHUNK
            pltpu.make_async_copy(kv.at[srcall.at[pl.ds(off, CHUNK)]], kb, sk).wait()
            pltpu.make_async_copy(q.at[dg], qbf, sq).wait()

            @pl.when(p > 0)
            def _():
                pltpu.make_async_copy(xb, sh_den.at[ds_], sx).wait()
                pltpu.make_async_copy(wb, sh_num.at[ds_], sw).wait()

            for i in range(CHUNK // 16):
                ds_[pl.ds(i * 16, 16)] = dstall[pl.ds(off + i * 16, 16)]

            @plsc.parallel_loop(0, CHUNK, 1, unroll=2 if D == DP else 4)
            def edge(e):
                for h in range(groups):
                    kvv = kb[e, pl.ds(h * 16, 16)]
                    qv = qbf[e, pl.ds(h * 16, 16)]
                    exv = jnp.exp(allsum(qv * kvv))
                    xb[e, pl.ds(h * 16, 16)] = exv
                    wb[e, pl.ds(h * 16, 16)] = exv * kb[e, pl.ds(D + h * 16, 16)]

            pltpu.async_copy(xb, sh_den.at[ds_], sx, add=True)
            pltpu.async_copy(wb, sh_num.at[ds_], sw, add=True)

        bufs0 = (dstg0, kvb0, qb0, skv0, sq0)
        bufs1 = (dstg1, kvb1, qb1, skv1, sq1)
        sc0 = (exb0, wvb0, dsts0, sx0, sw0)
        sc1 = (exb1, wvb1, dsts1, sx1, sw1)
        pairs = CHUNKS_PER_W // 2
        fire(0, *bufs0)

        def pair(p, _):
            fire(2 * p + 1, *bufs1)
            drain_compute_scatter(p, 2 * p, *bufs0, *sc0)

            @pl.when(p < pairs - 1)
            def _():
                fire(2 * p + 2, *bufs0)

            drain_compute_scatter(p, 2 * p + 1, *bufs1, *sc1)
            return 0

        lax.fori_loop(0, pairs, pair, 0)
        pltpu.make_async_copy(exb0, sh_den.at[dsts0], sx0).wait()
        pltpu.make_async_copy(wvb0, sh_num.at[dsts0], sw0).wait()
        pltpu.make_async_copy(exb1, sh_den.at[dsts1], sx1).wait()
        pltpu.make_async_copy(wvb1, sh_num.at[dsts1], sw1).wait()
        plsc.subcore_barrier()
        pltpu.sync_copy(sh_den.at[pl.ds(r0, rows)], out.at[c, 0, pl.ds(r0, rows)])
        pltpu.sync_copy(sh_num.at[pl.ds(r0, rows)], out.at[c, 1, pl.ds(r0, rows)])
        plsc.subcore_barrier()


def _edge_stage(kv, q, srcp, dstp, zer, D):
    meta = []
    ins = []
    for (s, d) in EDGE_TYPES:
        en = s + "__" + d
        ins.append(kv[en])
        meta.append((NODE_TYPES.index(d), N_NODES[d], N_PAD[d]))
    for t in NODE_TYPES:
        ins.append(q[t])
    for (s, d) in EDGE_TYPES:
        en = s + "__" + d
        ins += [srcp[en], dstp[en]]
    ins.append(zer)
    out_type = tuple(jax.ShapeDtypeStruct((2, 2, N_PAD[d], DP), jnp.float32) for (s, d) in EDGE_TYPES)
    mesh = plsc.VectorSubcoreMesh(core_axis_name="c", subcore_axis_name="s")
    f = pl.kernel(
        functools.partial(_edge_body, meta, D),
        out_type=out_type,
        mesh=mesh,
        scratch_types=[pltpu.VMEM((CHUNK,), jnp.int32)] * 4
        + [pltpu.VMEM((CHUNKS_PER_W * CHUNK,), jnp.int32)] * 2
        + [pltpu.VMEM((CHUNK, 2 * D), jnp.float32)] * 2
        + [pltpu.VMEM((CHUNK, DP), jnp.float32)] * 6
        + [pltpu.VMEM_SHARED((N_PAD["paper"], DP), jnp.float32)] * 2
        + [pltpu.SemaphoreType.DMA] * 10,
    )
    return f(*ins)


def _post_body(n_en, dout, slab0, *refs):
    slabs = (slab0,) + refs[:n_en - 1]
    wa_ref, ba_ref, out_ref = refs[n_en - 1:]
    agg = None
    for sl_ref in slabs:
        sl = sl_ref[...]
        den = sl[0, 0, :, :dout] + sl[1, 0, :, :dout]
        num = sl[0, 1, :, :dout] + sl[1, 1, :, :dout]
        part = num / (den + 1e-16)
        agg = part if agg is None else agg + part
    g = 0.5 * agg * (1.0 + jnp.tanh(0.7978845608028654 * (agg + 0.044715 * agg * agg * agg)))
    out_ref[...] = jnp.dot(g, wa_ref[...], preferred_element_type=jnp.float32) + ba_ref[...]


def _post(slabs, wa, ba, n):
    n_en = len(slabs)
    dout = wa.shape[1]
    R = 1000
    grid = n // R
    return pl.pallas_call(
        functools.partial(_post_body, n_en, dout),
        out_shape=jax.ShapeDtypeStruct((n, dout), jnp.float32),
        grid=(grid,),
        in_specs=[pl.BlockSpec((2, 2, R, DP), lambda i: (0, 0, i, 0))] * n_en
        + [pl.BlockSpec((dout, dout), lambda i: (0, 0)),
           pl.BlockSpec((1, dout), lambda i: (0, 0))],
        out_specs=pl.BlockSpec((R, dout), lambda i: (i, 0)),
    )(*slabs, wa, ba.reshape(1, dout))


def _final_body(h_ref, w_ref, b_ref, o_ref):
    logits = jnp.dot(h_ref[...], w_ref[...], preferred_element_type=jnp.float32) + b_ref[...]
    m = jnp.max(logits, axis=1, keepdims=True)
    e = jnp.exp(logits - m)
    o_ref[...] = e / jnp.sum(e, axis=1, keepdims=True)


def _final(h, lin_W, lin_b):
    n = h.shape[0]
    return pl.pallas_call(
        _final_body,
        out_shape=jax.ShapeDtypeStruct((n, OUT_DIM), jnp.float32),
        grid=(10,),
        in_specs=[
            pl.BlockSpec((n // 10, 64), lambda i: (i, 0)),
            pl.BlockSpec((64, OUT_DIM), lambda i: (0, 0)),
            pl.BlockSpec((1, OUT_DIM), lambda i: (0, 0)),
        ],
        out_specs=pl.BlockSpec((n // 10, OUT_DIM), lambda i: (i, 0)),
    )(h, lin_W, lin_b.reshape(1, OUT_DIM))


def _layer(xd, srcp, dstp, zer, p, L):
    din, dout = DIMS[L]
    dh = dout // H
    scale = 1.0 / np.sqrt(dh)
    src_of = {t: [en for en in EDGE_TYPES if en[0] == t] for t in NODE_TYPES}
    dst_of = {t: [en for en in EDGE_TYPES if en[1] == t] for t in NODE_TYPES}
    q = {}
    kv = {}
    for t in NODE_TYPES:
        arel_s = []
        mrel = []
        for (s, d) in src_of[t]:
            en = s + "__" + d
            arel_s.append(p["c%d_arel_%s" % (L, en)]
                          * (p["c%d_prel_%s" % (L, en)] * scale)[:, None, None])
            mrel.append(p["c%d_mrel_%s" % (L, en)])
        wf, bf = _prep(L, t, p, arel_s, mrel)
        outs = _proj(xd[t], wf, bf, len(src_of[t]), dout)
        q[t] = outs[0]
        for i, (s, d) in enumerate(src_of[t]):
            kv[s + "__" + d] = outs[1 + i]
    slabs = _edge_stage(kv, q, srcp, dstp, zer, dout)
    slab_of = {s + "__" + d: slabs[i] for i, (s, d) in enumerate(EDGE_TYPES)}
    out = {}
    for t in NODE_TYPES:
        sl = [slab_of[s + "__" + d] for (s, d) in dst_of[t]]
        out[t] = _post(sl, p["c%d_Wa_%s" % (L, t)], p["c%d_ba_%s" % (L, t)], N_NODES[t])
    return out


def kernel(x_paper, x_author, x_term, edge_index_paper__author, edge_index_author__paper, edge_index_paper__term, edge_index_term__paper, c1_Wk_paper, c1_Wq_paper, c1_Wv_paper, c1_bk_paper, c1_bq_paper, c1_bv_paper, c1_Wa_paper, c1_ba_paper, c1_Wk_author, c1_Wq_author, c1_Wv_author, c1_bk_author, c1_bq_author, c1_bv_author, c1_Wa_author, c1_ba_author, c1_Wk_term, c1_Wq_term, c1_Wv_term, c1_bk_term, c1_bq_term, c1_bv_term, c1_Wa_term, c1_ba_term, c1_arel_paper__author, c1_mrel_paper__author, c1_prel_paper__author, c1_arel_author__paper, c1_mrel_author__paper, c1_prel_author__paper, c1_arel_paper__term, c1_mrel_paper__term, c1_prel_paper__term, c1_arel_term__paper, c1_mrel_term__paper, c1_prel_term__paper, c2_Wk_paper, c2_Wq_paper, c2_Wv_paper, c2_bk_paper, c2_bq_paper, c2_bv_paper, c2_Wa_paper, c2_ba_paper, c2_Wk_author, c2_Wq_author, c2_Wv_author, c2_bk_author, c2_bq_author, c2_bv_author, c2_Wa_author, c2_ba_author, c2_Wk_term, c2_Wq_term, c2_Wv_term, c2_bk_term, c2_bq_term, c2_bv_term, c2_Wa_term, c2_ba_term, c2_arel_paper__author, c2_mrel_paper__author, c2_prel_paper__author, c2_arel_author__paper, c2_mrel_author__paper, c2_prel_author__paper, c2_arel_paper__term, c2_mrel_paper__term, c2_prel_paper__term, c2_arel_term__paper, c2_mrel_term__paper, c2_prel_term__paper, lin_W, lin_b):
    kwargs = dict(locals())
    p = {kk: vv for kk, vv in kwargs.items() if not kk.startswith("edge_index_")}
    srcp = {}
    dstp = {}
    for (s, d) in EDGE_TYPES:
        en = s + "__" + d
        ei = kwargs["edge_index_" + en]
        pad = E_PAD - N_EDGES
        srcp[en] = jnp.concatenate([ei[0], jnp.zeros((pad,), ei.dtype)])
        dstp[en] = jnp.concatenate([ei[1], jnp.full((pad,), N_NODES[d], ei.dtype)])
    zer = jnp.zeros((N_PAD["paper"], DP), jnp.float32)
    xd = {t: p["x_" + t] for t in NODE_TYPES}
    h1 = _layer(xd, srcp, dstp, zer, p, 1)
    h2 = _layer(h1, srcp, dstp, zer, p, 2)
    hcat = jnp.vstack([h2[t] for t in NODE_TYPES])
    return _final(hcat, lin_W, lin_b)
